# Initial kernel scaffold; baseline (speedup 1.0000x reference)
import jax
import jax.numpy as jnp
from jax.experimental import pallas as pl


def kernel(edge_index, edge_type, entity_ids, batch_ids, node_embeddings, W1, root1, b1, W2, root2, b2):
    raise NotImplementedError("write your pallas kernel here")


# speedup vs baseline: 11.9630x; 11.9630x over previous
import dataclasses

import jax
import jax.numpy as jnp
from jax import lax
from jax.experimental import pallas as pl
from jax.experimental.pallas import tpu as pltpu
from jax.experimental.pallas import tpu_sc as plsc

N = 10000
E = 320000
D = 128
R = 18
RP = R + 1
B = 128

NSC = 2
NSUB = 16
NW = NSC * NSUB
EPT = E // NW
KB = 80
NBLK = EPT // KB
CPP = 180224
CPT = CPP // NW
ZR = 80

_mesh = plsc.VectorSubcoreMesh(core_axis_name="c", subcore_axis_name="s")

_sc_params = pltpu.CompilerParams()
if "needs_layout_passes" in pltpu.CompilerParams.__dataclass_fields__:
    _sc_params = dataclasses.replace(_sc_params, needs_layout_passes=False)


def _zero_buf_rows(buf, rows):
    @pl.loop(0, rows)
    def _(i):
        for j in range(D // 16):
            buf[i, pl.ds(j * 16, 16)] = jnp.zeros((16,), jnp.float32)


def _gather_x(node_embeddings, entity_ids):
    @pl.kernel(
        out_type=jax.ShapeDtypeStruct((N, D), jnp.float32),
        mesh=_mesh,
        scratch_types=[
            pltpu.VMEM((KB,), jnp.int32),
            pltpu.VMEM((KB, D), jnp.float32),
            pltpu.SemaphoreType.DMA,
        ],
    )
    def k(tab_hbm, eid_hbm, x_hbm, idx_v, rows_v, sem):
        w = lax.axis_index("c") * NSUB + lax.axis_index("s")

        @pl.loop(w, N // KB, step=NW)
        def _(b):
            off = b * KB
            pltpu.sync_copy(eid_hbm.at[pl.ds(off, KB)], idx_v)
            pltpu.async_copy(tab_hbm.at[idx_v], rows_v, sem).wait()
            pltpu.sync_copy(rows_v, x_hbm.at[pl.ds(off, KB)])

    return k(node_embeddings, entity_ids)


def _edge_counts(dst, typ):
    @pl.kernel(
        out_type=jax.ShapeDtypeStruct((NSC, CPP), jnp.float32),
        mesh=_mesh,
        scratch_types=[
            pltpu.VMEM((KB,), jnp.int32),
            pltpu.VMEM((KB,), jnp.int32),
            pltpu.VMEM((KB,), jnp.int32),
            pltpu.VMEM((KB,), jnp.float32),
            pltpu.VMEM((CPT,), jnp.float32),
            pltpu.VMEM_SHARED((CPP,), jnp.float32),
        ],
    )
    def k(dst_hbm, typ_hbm, out_hbm, dst_v, typ_v, key_v, ones_v, z_v, acc_sh):
        c = lax.axis_index("c")
        s = lax.axis_index("s")
        w = c * NSUB + s

        for j in range(KB // 16):
            ones_v[pl.ds(j * 16, 16)] = jnp.ones((16,), jnp.float32)

        @pl.loop(0, CPT, step=16)
        def _(i):
            z_v[pl.ds(i, 16)] = jnp.zeros((16,), jnp.float32)

        pltpu.sync_copy(z_v, acc_sh.at[pl.ds(s * CPT * 2, CPT)])
        pltpu.sync_copy(z_v, acc_sh.at[pl.ds(s * CPT * 2 + CPT, CPT)])
        plsc.subcore_barrier()

        base = w * EPT

        @pl.loop(0, NBLK)
        def _(i):
            off = base + i * KB
            pltpu.sync_copy(dst_hbm.at[pl.ds(off, KB)], dst_v)
            pltpu.sync_copy(typ_hbm.at[pl.ds(off, KB)], typ_v)
            for j in range(KB // 16):
                sl = pl.ds(j * 16, 16)
                key_v[sl] = dst_v[sl] * R + typ_v[sl]
            pltpu.sync_copy(ones_v, acc_sh.at[key_v], add=True)

        plsc.subcore_barrier()
        pltpu.sync_copy(acc_sh.at[pl.ds(s * CPT * 2, CPT * 2)],
                        out_hbm.at[c, pl.ds(s * CPT * 2, CPT * 2)])

    return k(dst, typ)


def _inv_counts(cnt):
    @pl.kernel(
        out_type=jax.ShapeDtypeStruct((CPP,), jnp.float32),
        mesh=_mesh,
        scratch_types=[
            pltpu.VMEM((CPT,), jnp.float32),
            pltpu.VMEM((CPT,), jnp.float32),
        ],
    )
    def k(cnt_hbm, out_hbm, a_v, b_v):
        w = lax.axis_index("c") * NSUB + lax.axis_index("s")
        base = w * CPT
        pltpu.sync_copy(cnt_hbm.at[0, pl.ds(base, CPT)], a_v)
        pltpu.sync_copy(cnt_hbm.at[1, pl.ds(base, CPT)], b_v)

        @pl.loop(0, CPT, step=16)
        def _(i):
            sl = pl.ds(i, 16)
            a_v[sl] = 1.0 / jnp.maximum(a_v[sl] + b_v[sl], 1.0)

        pltpu.sync_copy(a_v, out_hbm.at[pl.ds(base, CPT)])

    return k(cnt)


def _edge_aggregate(src, dst, typ, inv, yflat):
    @pl.kernel(
        out_type=jax.ShapeDtypeStruct((NSC, N, D), jnp.float32),
        mesh=_mesh,
        compiler_params=_sc_params,
        scratch_types=[
            pltpu.VMEM((KB,), jnp.int32),
            pltpu.VMEM((KB,), jnp.int32),
            pltpu.VMEM((KB,), jnp.int32),
            pltpu.VMEM((KB,), jnp.int32),
            pltpu.VMEM((KB,), jnp.int32),
            pltpu.VMEM((KB + 16,), jnp.float32),
            pltpu.VMEM((KB, D), jnp.float32),
            pltpu.VMEM((ZR, D), jnp.float32),
            pltpu.VMEM_SHARED((N, D), jnp.float32),
            pltpu.SemaphoreType.DMA,
            pltpu.SemaphoreType.DMA,
        ],
    )
    def k(src_hbm, dst_hbm, typ_hbm, inv_hbm, y_hbm, out_hbm,
          src_v, dst_v, typ_v, k1_v, k2_v, w_v, rows_v, z_v, acc_sh,
          sem1, sem2):
        c = lax.axis_index("c")
        s = lax.axis_index("s")
        w = c * NSUB + s

        _zero_buf_rows(z_v, ZR)

        @pl.loop(s, N // ZR, step=NSUB)
        def _(g):
            pltpu.sync_copy(z_v, acc_sh.at[pl.ds(g * ZR, ZR)])

        plsc.subcore_barrier()

        base = w * EPT

        @pl.loop(0, NBLK)
        def _(i):
            off = base + i * KB
            pltpu.sync_copy(src_hbm.at[pl.ds(off, KB)], src_v)
            pltpu.sync_copy(dst_hbm.at[pl.ds(off, KB)], dst_v)
            pltpu.sync_copy(typ_hbm.at[pl.ds(off, KB)], typ_v)
            for j in range(KB // 16):
                sl = pl.ds(j * 16, 16)
                t16 = typ_v[sl]
                k1_v[sl] = t16 * N + src_v[sl]
                k2_v[sl] = dst_v[sl] * R + t16
            cp1 = pltpu.async_copy(y_hbm.at[k1_v], rows_v, sem1)
            cp2 = pltpu.async_copy(inv_hbm.at[k2_v], w_v.at[pl.ds(16, KB)],
                                   sem2)
            cp2.wait()
            cp1.wait()
            for e in range(KB):
                wv = plsc.load_gather(
                    w_v, [jnp.full((16,), 16 + e, jnp.int32)])
                for j in range(D // 16):
                    sl = pl.ds(j * 16, 16)
                    rows_v[e, sl] = rows_v[e, sl] * wv
            pltpu.sync_copy(rows_v, acc_sh.at[dst_v], add=True)

        plsc.subcore_barrier()

        @pl.loop(s, N // ZR, step=NSUB)
        def _(g):
            pltpu.sync_copy(acc_sh.at[pl.ds(g * ZR, ZR)],
                            out_hbm.at[c, pl.ds(g * ZR, ZR)])

    return k(src, dst, typ, inv, yflat)


BN = 400


def _y_from_x(x, wall):
    def body(x_ref, w_ref, o_ref):
        o_ref[0] = jnp.dot(x_ref[...], w_ref[0],
                           preferred_element_type=jnp.float32)

    return pl.pallas_call(
        body,
        grid=(N // BN, RP),
        in_specs=[
            pl.BlockSpec((BN, D), lambda i, r: (i, 0)),
            pl.BlockSpec((1, D, D), lambda i, r: (r, 0, 0)),
        ],
        out_specs=pl.BlockSpec((1, BN, D), lambda i, r: (r, i, 0)),
        out_shape=jax.ShapeDtypeStruct((RP, N, D), jnp.float32),
    )(x, wall)


def _y_from_agg(agg0, agg1, yprev, bias, wall):
    def body(a0_ref, a1_ref, yr_ref, b_ref, w_ref, o_ref):
        h = a0_ref[...] + a1_ref[...] + yr_ref[0] + b_ref[0][None, :]
        o_ref[0] = jnp.dot(h, w_ref[0],
                           preferred_element_type=jnp.float32)

    return pl.pallas_call(
        body,
        grid=(N // BN, RP),
        in_specs=[
            pl.BlockSpec((BN, D), lambda i, r: (i, 0)),
            pl.BlockSpec((BN, D), lambda i, r: (i, 0)),
            pl.BlockSpec((1, BN, D), lambda i, r: (R, i, 0)),
            pl.BlockSpec((1, D), lambda i, r: (0, 0)),
            pl.BlockSpec((1, D, D), lambda i, r: (r, 0, 0)),
        ],
        out_specs=pl.BlockSpec((1, BN, D), lambda i, r: (r, i, 0)),
        out_shape=jax.ShapeDtypeStruct((RP, N, D), jnp.float32),
    )(agg0, agg1, yprev, bias, wall)


def _batch_pool(agg0, agg1, yprev, bias, bid3):
    def body(a0_ref, a1_ref, yr_ref, b_ref, bid_ref, o_ref):
        h = a0_ref[...] + a1_ref[...] + yr_ref[0] + b_ref[0][None, :]
        bb = bid_ref[0, 0, :]
        oh = (lax.broadcasted_iota(jnp.int32, (B, BN), 0)
              == bb[None, :]).astype(jnp.float32)

        @pl.when(pl.program_id(0) == 0)
        def _():
            o_ref[...] = jnp.zeros_like(o_ref)

        o_ref[...] += jnp.dot(oh, h, preferred_element_type=jnp.float32)

    return pl.pallas_call(
        body,
        grid=(N // BN,),
        in_specs=[
            pl.BlockSpec((BN, D), lambda i: (i, 0)),
            pl.BlockSpec((BN, D), lambda i: (i, 0)),
            pl.BlockSpec((1, BN, D), lambda i: (R, i, 0)),
            pl.BlockSpec((1, D), lambda i: (0, 0)),
            pl.BlockSpec((1, 1, BN), lambda i: (i, 0, 0)),
        ],
        out_specs=pl.BlockSpec((B, D), lambda i: (0, 0)),
        out_shape=jax.ShapeDtypeStruct((B, D), jnp.float32),
    )(agg0, agg1, yprev, bias, bid3)


def kernel(edge_index, edge_type, entity_ids, batch_ids, node_embeddings,
           W1, root1, b1, W2, root2, b2):
    src = edge_index[0]
    dst = edge_index[1]
    typ = edge_type.astype(jnp.int32)

    x = _gather_x(node_embeddings, entity_ids.astype(jnp.int32))
    cnt = _edge_counts(dst, typ)
    inv = _inv_counts(cnt)

    w1a = jnp.concatenate([W1, root1[None]], axis=0)
    w2a = jnp.concatenate([W2, root2[None]], axis=0)

    y1 = _y_from_x(x, w1a)
    agg1 = _edge_aggregate(src, dst, typ, inv, y1.reshape(N * RP, D))
    y2 = _y_from_agg(agg1[0], agg1[1], y1, b1.reshape(1, D), w2a)
    agg2 = _edge_aggregate(src, dst, typ, inv, y2.reshape(N * RP, D))
    out = _batch_pool(agg2[0], agg2[1], y2, b2.reshape(1, D),
                      batch_ids.astype(jnp.int32).reshape(N // BN, 1, BN))
    return out


# speedup vs baseline: 14.2128x; 1.1881x over previous
import dataclasses

import jax
import jax.numpy as jnp
from jax import lax
from jax.experimental import pallas as pl
from jax.experimental.pallas import tpu as pltpu
from jax.experimental.pallas import tpu_sc as plsc

N = 10000
E = 320000
D = 128
R = 18
RP = R + 1
B = 128

NSC = 2
NSUB = 16
NW = NSC * NSUB
EPT = E // NW
KB = 80
NBLK = EPT // KB
CPP = 180224
CPT = CPP // NW
ZR = 80

_mesh = plsc.VectorSubcoreMesh(core_axis_name="c", subcore_axis_name="s")

_sc_params = pltpu.CompilerParams()
if "needs_layout_passes" in pltpu.CompilerParams.__dataclass_fields__:
    _sc_params = dataclasses.replace(_sc_params, needs_layout_passes=False)


def _zero_buf_rows(buf, rows):
    @pl.loop(0, rows)
    def _(i):
        for j in range(D // 16):
            buf[i, pl.ds(j * 16, 16)] = jnp.zeros((16,), jnp.float32)


def _gather_x(node_embeddings, entity_ids):
    @pl.kernel(
        out_type=jax.ShapeDtypeStruct((N, D), jnp.float32),
        mesh=_mesh,
        scratch_types=[
            pltpu.VMEM((KB,), jnp.int32),
            pltpu.VMEM((KB, D), jnp.float32),
            pltpu.SemaphoreType.DMA,
        ],
    )
    def k(tab_hbm, eid_hbm, x_hbm, idx_v, rows_v, sem):
        w = lax.axis_index("c") * NSUB + lax.axis_index("s")

        @pl.loop(w, N // KB, step=NW)
        def _(b):
            off = b * KB
            pltpu.sync_copy(eid_hbm.at[pl.ds(off, KB)], idx_v)
            pltpu.async_copy(tab_hbm.at[idx_v], rows_v, sem).wait()
            pltpu.sync_copy(rows_v, x_hbm.at[pl.ds(off, KB)])

    return k(node_embeddings, entity_ids)


def _edge_counts(dst, typ):
    @pl.kernel(
        out_type=jax.ShapeDtypeStruct((NSC, CPP), jnp.float32),
        mesh=_mesh,
        scratch_types=[
            pltpu.VMEM((KB,), jnp.int32),
            pltpu.VMEM((KB,), jnp.int32),
            pltpu.VMEM((KB,), jnp.int32),
            pltpu.VMEM((KB,), jnp.float32),
            pltpu.VMEM((CPT,), jnp.float32),
            pltpu.VMEM_SHARED((CPP,), jnp.float32),
        ],
    )
    def k(dst_hbm, typ_hbm, out_hbm, dst_v, typ_v, key_v, ones_v, z_v, acc_sh):
        c = lax.axis_index("c")
        s = lax.axis_index("s")
        w = c * NSUB + s

        for j in range(KB // 16):
            ones_v[pl.ds(j * 16, 16)] = jnp.ones((16,), jnp.float32)

        @pl.loop(0, CPT, step=16)
        def _(i):
            z_v[pl.ds(i, 16)] = jnp.zeros((16,), jnp.float32)

        pltpu.sync_copy(z_v, acc_sh.at[pl.ds(s * CPT * 2, CPT)])
        pltpu.sync_copy(z_v, acc_sh.at[pl.ds(s * CPT * 2 + CPT, CPT)])
        plsc.subcore_barrier()

        base = w * EPT

        @pl.loop(0, NBLK)
        def _(i):
            off = base + i * KB
            pltpu.sync_copy(dst_hbm.at[pl.ds(off, KB)], dst_v)
            pltpu.sync_copy(typ_hbm.at[pl.ds(off, KB)], typ_v)
            for j in range(KB // 16):
                sl = pl.ds(j * 16, 16)
                key_v[sl] = dst_v[sl] * R + typ_v[sl]
            pltpu.sync_copy(ones_v, acc_sh.at[key_v], add=True)

        plsc.subcore_barrier()
        pltpu.sync_copy(acc_sh.at[pl.ds(s * CPT * 2, CPT * 2)],
                        out_hbm.at[c, pl.ds(s * CPT * 2, CPT * 2)])

    return k(dst, typ)


def _inv_counts(cnt):
    @pl.kernel(
        out_type=jax.ShapeDtypeStruct((CPP,), jnp.float32),
        mesh=_mesh,
        scratch_types=[
            pltpu.VMEM((CPT,), jnp.float32),
            pltpu.VMEM((CPT,), jnp.float32),
        ],
    )
    def k(cnt_hbm, out_hbm, a_v, b_v):
        w = lax.axis_index("c") * NSUB + lax.axis_index("s")
        base = w * CPT
        pltpu.sync_copy(cnt_hbm.at[0, pl.ds(base, CPT)], a_v)
        pltpu.sync_copy(cnt_hbm.at[1, pl.ds(base, CPT)], b_v)

        @pl.loop(0, CPT, step=16)
        def _(i):
            sl = pl.ds(i, 16)
            a_v[sl] = 1.0 / jnp.maximum(a_v[sl] + b_v[sl], 1.0)

        pltpu.sync_copy(a_v, out_hbm.at[pl.ds(base, CPT)])

    return k(cnt)


def _edge_aggregate(src, dst, typ, inv, yflat):
    @pl.kernel(
        out_type=jax.ShapeDtypeStruct((NSC, N, D), jnp.float32),
        mesh=_mesh,
        compiler_params=_sc_params,
        scratch_types=[
            pltpu.VMEM((KB,), jnp.int32),
            pltpu.VMEM((KB,), jnp.int32),
            pltpu.VMEM((KB,), jnp.int32),
            pltpu.VMEM((KB,), jnp.int32),
            pltpu.VMEM((KB,), jnp.int32),
            pltpu.VMEM((KB + 16,), jnp.float32),
            pltpu.VMEM((KB, D), jnp.float32),
            pltpu.VMEM((ZR, D), jnp.float32),
            pltpu.VMEM_SHARED((N, D), jnp.float32),
            pltpu.SemaphoreType.DMA,
            pltpu.SemaphoreType.DMA,
        ],
    )
    def k(src_hbm, dst_hbm, typ_hbm, inv_hbm, y_hbm, out_hbm,
          src_v, dst_v, typ_v, k1_v, k2_v, w_v, rows_v, z_v, acc_sh,
          sem1, sem2):
        c = lax.axis_index("c")
        s = lax.axis_index("s")
        w = c * NSUB + s

        _zero_buf_rows(z_v, ZR)

        @pl.loop(s, N // ZR, step=NSUB)
        def _(g):
            pltpu.sync_copy(z_v, acc_sh.at[pl.ds(g * ZR, ZR)])

        plsc.subcore_barrier()

        base = w * EPT

        @pl.loop(0, NBLK)
        def _(i):
            off = base + i * KB
            pltpu.sync_copy(src_hbm.at[pl.ds(off, KB)], src_v)
            pltpu.sync_copy(dst_hbm.at[pl.ds(off, KB)], dst_v)
            pltpu.sync_copy(typ_hbm.at[pl.ds(off, KB)], typ_v)
            for j in range(KB // 16):
                sl = pl.ds(j * 16, 16)
                t16 = typ_v[sl]
                k1_v[sl] = src_v[sl] * RP + t16
                k2_v[sl] = dst_v[sl] * R + t16
            cp1 = pltpu.async_copy(y_hbm.at[k1_v], rows_v, sem1)
            cp2 = pltpu.async_copy(inv_hbm.at[k2_v], w_v.at[pl.ds(16, KB)],
                                   sem2)
            cp2.wait()
            cp1.wait()
            for e in range(KB):
                wv = plsc.load_gather(
                    w_v, [jnp.full((16,), 16 + e, jnp.int32)])
                for j in range(D // 16):
                    sl = pl.ds(j * 16, 16)
                    rows_v[e, sl] = rows_v[e, sl] * wv
            pltpu.sync_copy(rows_v, acc_sh.at[dst_v], add=True)

        plsc.subcore_barrier()

        @pl.loop(s, N // ZR, step=NSUB)
        def _(g):
            pltpu.sync_copy(acc_sh.at[pl.ds(g * ZR, ZR)],
                            out_hbm.at[c, pl.ds(g * ZR, ZR)])

    return k(src, dst, typ, inv, yflat)


BN = 400


def _y_from_x(x, wall):
    def body(x_ref, w_ref, o_ref):
        o_ref[...] = jnp.dot(
            x_ref[...], w_ref[...],
            preferred_element_type=jnp.float32).reshape(BN, RP, D)

    return pl.pallas_call(
        body,
        grid=(N // BN,),
        in_specs=[
            pl.BlockSpec((BN, D), lambda i: (i, 0)),
            pl.BlockSpec((D, RP * D), lambda i: (0, 0)),
        ],
        out_specs=pl.BlockSpec((BN, RP, D), lambda i: (i, 0, 0)),
        out_shape=jax.ShapeDtypeStruct((N, RP, D), jnp.float32),
    )(x, wall)


def _y_from_agg(agg0, agg1, yprev, bias, wall):
    def body(a0_ref, a1_ref, yr_ref, b_ref, w_ref, o_ref):
        h = a0_ref[...] + a1_ref[...] + yr_ref[...] + b_ref[0][None, :]
        o_ref[...] = jnp.dot(
            h, w_ref[...],
            preferred_element_type=jnp.float32).reshape(BN, RP, D)

    return pl.pallas_call(
        body,
        grid=(N // BN,),
        in_specs=[
            pl.BlockSpec((BN, D), lambda i: (i, 0)),
            pl.BlockSpec((BN, D), lambda i: (i, 0)),
            pl.BlockSpec((BN, D), lambda i: (i, 0)),
            pl.BlockSpec((1, D), lambda i: (0, 0)),
            pl.BlockSpec((D, RP * D), lambda i: (0, 0)),
        ],
        out_specs=pl.BlockSpec((BN, RP, D), lambda i: (i, 0, 0)),
        out_shape=jax.ShapeDtypeStruct((N, RP, D), jnp.float32),
    )(agg0, agg1, yprev, bias, wall)


def _batch_pool(agg0, agg1, yprev, bias, bid3):
    def body(a0_ref, a1_ref, yr_ref, b_ref, bid_ref, o_ref):
        h = a0_ref[...] + a1_ref[...] + yr_ref[...] + b_ref[0][None, :]
        bb = bid_ref[0, 0, :]
        oh = (lax.broadcasted_iota(jnp.int32, (B, BN), 0)
              == bb[None, :]).astype(jnp.float32)

        @pl.when(pl.program_id(0) == 0)
        def _():
            o_ref[...] = jnp.zeros_like(o_ref)

        o_ref[...] += jnp.dot(oh, h, preferred_element_type=jnp.float32)

    return pl.pallas_call(
        body,
        grid=(N // BN,),
        in_specs=[
            pl.BlockSpec((BN, D), lambda i: (i, 0)),
            pl.BlockSpec((BN, D), lambda i: (i, 0)),
            pl.BlockSpec((BN, D), lambda i: (i, 0)),
            pl.BlockSpec((1, D), lambda i: (0, 0)),
            pl.BlockSpec((1, 1, BN), lambda i: (i, 0, 0)),
        ],
        out_specs=pl.BlockSpec((B, D), lambda i: (0, 0)),
        out_shape=jax.ShapeDtypeStruct((B, D), jnp.float32),
    )(agg0, agg1, yprev, bias, bid3)


def kernel(edge_index, edge_type, entity_ids, batch_ids, node_embeddings,
           W1, root1, b1, W2, root2, b2):
    src = edge_index[0]
    dst = edge_index[1]
    typ = edge_type.astype(jnp.int32)

    x = _gather_x(node_embeddings, entity_ids.astype(jnp.int32))
    cnt = _edge_counts(dst, typ)
    inv = _inv_counts(cnt)

    w1a = jnp.concatenate([W1, root1[None]], axis=0)
    w2a = jnp.concatenate([W2, root2[None]], axis=0)
    w1f = w1a.transpose(1, 0, 2).reshape(D, RP * D)
    w2f = w2a.transpose(1, 0, 2).reshape(D, RP * D)

    y1 = _y_from_x(x, w1f)
    agg1 = _edge_aggregate(src, dst, typ, inv, y1.reshape(N * RP, D))
    y2 = _y_from_agg(agg1[0], agg1[1], y1[:, R, :], b1.reshape(1, D), w2f)
    agg2 = _edge_aggregate(src, dst, typ, inv, y2.reshape(N * RP, D))
    out = _batch_pool(agg2[0], agg2[1], y2[:, R, :], b2.reshape(1, D),
                      batch_ids.astype(jnp.int32).reshape(N // BN, 1, BN))
    return out


# speedup vs baseline: 14.8630x; 1.0457x over previous
import dataclasses

import jax
import jax.numpy as jnp
from jax import lax
from jax.experimental import pallas as pl
from jax.experimental.pallas import tpu as pltpu
from jax.experimental.pallas import tpu_sc as plsc

N = 10000
E = 320000
D = 128
R = 18
RP = R + 1
B = 128

NSC = 2
NSUB = 16
NW = NSC * NSUB
EPT = E // NW
KB = 80
NBLK = EPT // KB
CPP = 180224
CPT = CPP // NW
ZR = 80

_mesh = plsc.VectorSubcoreMesh(core_axis_name="c", subcore_axis_name="s")

_sc_params = pltpu.CompilerParams()
if "needs_layout_passes" in pltpu.CompilerParams.__dataclass_fields__:
    _sc_params = dataclasses.replace(_sc_params, needs_layout_passes=False)


def _zero_buf_rows(buf, rows):
    @pl.loop(0, rows)
    def _(i):
        for j in range(D // 16):
            buf[i, pl.ds(j * 16, 16)] = jnp.zeros((16,), jnp.float32)


def _gather_x(node_embeddings, entity_ids):
    @pl.kernel(
        out_type=jax.ShapeDtypeStruct((N, D), jnp.float32),
        mesh=_mesh,
        scratch_types=[
            pltpu.VMEM((KB,), jnp.int32),
            pltpu.VMEM((KB, D), jnp.float32),
            pltpu.SemaphoreType.DMA,
        ],
    )
    def k(tab_hbm, eid_hbm, x_hbm, idx_v, rows_v, sem):
        w = lax.axis_index("c") * NSUB + lax.axis_index("s")

        @pl.loop(w, N // KB, step=NW)
        def _(b):
            off = b * KB
            pltpu.sync_copy(eid_hbm.at[pl.ds(off, KB)], idx_v)
            pltpu.async_copy(tab_hbm.at[idx_v], rows_v, sem).wait()
            pltpu.sync_copy(rows_v, x_hbm.at[pl.ds(off, KB)])

    return k(node_embeddings, entity_ids)


def _edge_counts(dst, typ):
    @pl.kernel(
        out_type=jax.ShapeDtypeStruct((NSC, CPP), jnp.float32),
        mesh=_mesh,
        scratch_types=[
            pltpu.VMEM((KB,), jnp.int32),
            pltpu.VMEM((KB,), jnp.int32),
            pltpu.VMEM((KB,), jnp.int32),
            pltpu.VMEM((KB,), jnp.float32),
            pltpu.VMEM((CPT,), jnp.float32),
            pltpu.VMEM_SHARED((CPP,), jnp.float32),
        ],
    )
    def k(dst_hbm, typ_hbm, out_hbm, dst_v, typ_v, key_v, ones_v, z_v, acc_sh):
        c = lax.axis_index("c")
        s = lax.axis_index("s")
        w = c * NSUB + s

        for j in range(KB // 16):
            ones_v[pl.ds(j * 16, 16)] = jnp.ones((16,), jnp.float32)

        @pl.loop(0, CPT, step=16)
        def _(i):
            z_v[pl.ds(i, 16)] = jnp.zeros((16,), jnp.float32)

        pltpu.sync_copy(z_v, acc_sh.at[pl.ds(s * CPT * 2, CPT)])
        pltpu.sync_copy(z_v, acc_sh.at[pl.ds(s * CPT * 2 + CPT, CPT)])
        plsc.subcore_barrier()

        base = w * EPT

        @pl.loop(0, NBLK)
        def _(i):
            off = base + i * KB
            pltpu.sync_copy(dst_hbm.at[pl.ds(off, KB)], dst_v)
            pltpu.sync_copy(typ_hbm.at[pl.ds(off, KB)], typ_v)
            for j in range(KB // 16):
                sl = pl.ds(j * 16, 16)
                key_v[sl] = dst_v[sl] * R + typ_v[sl]
            pltpu.sync_copy(ones_v, acc_sh.at[key_v], add=True)

        plsc.subcore_barrier()
        pltpu.sync_copy(acc_sh.at[pl.ds(s * CPT * 2, CPT * 2)],
                        out_hbm.at[c, pl.ds(s * CPT * 2, CPT * 2)])

    return k(dst, typ)


def _inv_counts(cnt):
    @pl.kernel(
        out_type=jax.ShapeDtypeStruct((CPP,), jnp.float32),
        mesh=_mesh,
        scratch_types=[
            pltpu.VMEM((CPT,), jnp.float32),
            pltpu.VMEM((CPT,), jnp.float32),
        ],
    )
    def k(cnt_hbm, out_hbm, a_v, b_v):
        w = lax.axis_index("c") * NSUB + lax.axis_index("s")
        base = w * CPT
        pltpu.sync_copy(cnt_hbm.at[0, pl.ds(base, CPT)], a_v)
        pltpu.sync_copy(cnt_hbm.at[1, pl.ds(base, CPT)], b_v)

        @pl.loop(0, CPT, step=16)
        def _(i):
            sl = pl.ds(i, 16)
            a_v[sl] = 1.0 / jnp.maximum(a_v[sl] + b_v[sl], 1.0)

        pltpu.sync_copy(a_v, out_hbm.at[pl.ds(base, CPT)])

    return k(cnt)


def _edge_aggregate(pk, inv, yflat):
    @pl.kernel(
        out_type=jax.ShapeDtypeStruct((NSC, N, D), jnp.float32),
        mesh=_mesh,
        compiler_params=_sc_params,
        scratch_types=[
            pltpu.VMEM((3, KB), jnp.int32),
            pltpu.VMEM((3, KB), jnp.int32),
            pltpu.VMEM((KB,), jnp.int32),
            pltpu.VMEM((KB,), jnp.int32),
            pltpu.VMEM((KB,), jnp.int32),
            pltpu.VMEM((KB,), jnp.int32),
            pltpu.VMEM((KB + 16,), jnp.float32),
            pltpu.VMEM((KB + 16,), jnp.float32),
            pltpu.VMEM((KB, D), jnp.float32),
            pltpu.VMEM((KB, D), jnp.float32),
            pltpu.VMEM((ZR, D), jnp.float32),
            pltpu.VMEM_SHARED((N, D), jnp.float32),
            pltpu.SemaphoreType.DMA,
            pltpu.SemaphoreType.DMA,
            pltpu.SemaphoreType.DMA,
            pltpu.SemaphoreType.DMA,
        ],
    )
    def k(pk_hbm, inv_hbm, y_hbm, out_hbm,
          pk0, pk1, k1a, k1b, k2a, k2b, wa, wb, ra, rb, z_v, acc_sh,
          semr0, semr1, semi0, semi1):
        c = lax.axis_index("c")
        s = lax.axis_index("s")
        w = c * NSUB + s

        _zero_buf_rows(z_v, ZR)

        @pl.loop(s, N // ZR, step=NSUB)
        def _(g):
            pltpu.sync_copy(z_v, acc_sh.at[pl.ds(g * ZR, ZR)])

        plsc.subcore_barrier()

        bufs = ((pk0, k1a, k2a, wa, ra, semr0, semi0),
                (pk1, k1b, k2b, wb, rb, semr1, semi1))
        base = w * NBLK

        def load_and_fire(b, t):
            pk_v, k1_v, k2_v, w_v, rows_v, semr, semi = bufs[t]
            pltpu.sync_copy(pk_hbm.at[b], pk_v)
            for j in range(KB // 16):
                sl = pl.ds(j * 16, 16)
                t16 = pk_v[2, sl]
                k1_v[sl] = pk_v[0, sl] * RP + t16
                k2_v[sl] = pk_v[1, sl] * R + t16
            pltpu.async_copy(y_hbm.at[k1_v], rows_v, semr)
            pltpu.async_copy(inv_hbm.at[k2_v], w_v.at[pl.ds(16, KB)], semi)

        def drain_and_scatter(t):
            pk_v, k1_v, k2_v, w_v, rows_v, semr, semi = bufs[t]
            pltpu.make_async_copy(y_hbm.at[k1_v], rows_v, semr).wait()
            pltpu.make_async_copy(inv_hbm.at[k2_v], w_v.at[pl.ds(16, KB)],
                                  semi).wait()
            for e in range(KB):
                wv = plsc.load_gather(
                    w_v, [jnp.full((16,), 16 + e, jnp.int32)])
                for j in range(D // 16):
                    sl = pl.ds(j * 16, 16)
                    rows_v[e, sl] = rows_v[e, sl] * wv
            pltpu.sync_copy(rows_v, acc_sh.at[pk_v.at[1]], add=True)

        load_and_fire(base, 0)
        load_and_fire(base + 1, 1)

        @pl.loop(0, (NBLK - 1) // 2)
        def _(p):
            drain_and_scatter(0)
            load_and_fire(base + 2 * p + 2, 0)
            drain_and_scatter(1)

            @pl.when(2 * p + 3 < NBLK)
            def _():
                load_and_fire(base + 2 * p + 3, 1)

        drain_and_scatter(0)

        plsc.subcore_barrier()

        @pl.loop(s, N // ZR, step=NSUB)
        def _(g):
            pltpu.sync_copy(acc_sh.at[pl.ds(g * ZR, ZR)],
                            out_hbm.at[c, pl.ds(g * ZR, ZR)])

    return k(pk, inv, yflat)


BN = 400


def _y_from_x(x, wall):
    def body(x_ref, w_ref, o_ref):
        o_ref[...] = jnp.dot(
            x_ref[...], w_ref[...],
            preferred_element_type=jnp.float32).reshape(BN, RP, D)

    return pl.pallas_call(
        body,
        grid=(N // BN,),
        in_specs=[
            pl.BlockSpec((BN, D), lambda i: (i, 0)),
            pl.BlockSpec((D, RP * D), lambda i: (0, 0)),
        ],
        out_specs=pl.BlockSpec((BN, RP, D), lambda i: (i, 0, 0)),
        out_shape=jax.ShapeDtypeStruct((N, RP, D), jnp.float32),
    )(x, wall)


def _y_from_agg(agg0, agg1, yprev, bias, wall):
    def body(a0_ref, a1_ref, yr_ref, b_ref, w_ref, o_ref):
        h = a0_ref[...] + a1_ref[...] + yr_ref[...] + b_ref[0][None, :]
        o_ref[...] = jnp.dot(
            h, w_ref[...],
            preferred_element_type=jnp.float32).reshape(BN, RP, D)

    return pl.pallas_call(
        body,
        grid=(N // BN,),
        in_specs=[
            pl.BlockSpec((BN, D), lambda i: (i, 0)),
            pl.BlockSpec((BN, D), lambda i: (i, 0)),
            pl.BlockSpec((BN, D), lambda i: (i, 0)),
            pl.BlockSpec((1, D), lambda i: (0, 0)),
            pl.BlockSpec((D, RP * D), lambda i: (0, 0)),
        ],
        out_specs=pl.BlockSpec((BN, RP, D), lambda i: (i, 0, 0)),
        out_shape=jax.ShapeDtypeStruct((N, RP, D), jnp.float32),
    )(agg0, agg1, yprev, bias, wall)


def _batch_pool(agg0, agg1, yprev, bias, bid3):
    def body(a0_ref, a1_ref, yr_ref, b_ref, bid_ref, o_ref):
        h = a0_ref[...] + a1_ref[...] + yr_ref[...] + b_ref[0][None, :]
        bb = bid_ref[0, 0, :]
        oh = (lax.broadcasted_iota(jnp.int32, (B, BN), 0)
              == bb[None, :]).astype(jnp.float32)

        @pl.when(pl.program_id(0) == 0)
        def _():
            o_ref[...] = jnp.zeros_like(o_ref)

        o_ref[...] += jnp.dot(oh, h, preferred_element_type=jnp.float32)

    return pl.pallas_call(
        body,
        grid=(N // BN,),
        in_specs=[
            pl.BlockSpec((BN, D), lambda i: (i, 0)),
            pl.BlockSpec((BN, D), lambda i: (i, 0)),
            pl.BlockSpec((BN, D), lambda i: (i, 0)),
            pl.BlockSpec((1, D), lambda i: (0, 0)),
            pl.BlockSpec((1, 1, BN), lambda i: (i, 0, 0)),
        ],
        out_specs=pl.BlockSpec((B, D), lambda i: (0, 0)),
        out_shape=jax.ShapeDtypeStruct((B, D), jnp.float32),
    )(agg0, agg1, yprev, bias, bid3)


def kernel(edge_index, edge_type, entity_ids, batch_ids, node_embeddings,
           W1, root1, b1, W2, root2, b2):
    src = edge_index[0]
    dst = edge_index[1]
    typ = edge_type.astype(jnp.int32)
    pk = jnp.stack([src, dst, typ]).reshape(3, E // KB, KB).transpose(1, 0, 2)

    x = _gather_x(node_embeddings, entity_ids.astype(jnp.int32))
    cnt = _edge_counts(dst, typ)
    inv = _inv_counts(cnt)

    w1a = jnp.concatenate([W1, root1[None]], axis=0)
    w2a = jnp.concatenate([W2, root2[None]], axis=0)
    w1f = w1a.transpose(1, 0, 2).reshape(D, RP * D)
    w2f = w2a.transpose(1, 0, 2).reshape(D, RP * D)

    y1 = _y_from_x(x, w1f)
    agg1 = _edge_aggregate(pk, inv, y1.reshape(N * RP, D))
    y2 = _y_from_agg(agg1[0], agg1[1], y1[:, R, :], b1.reshape(1, D), w2f)
    agg2 = _edge_aggregate(pk, inv, y2.reshape(N * RP, D))
    out = _batch_pool(agg2[0], agg2[1], y2[:, R, :], b2.reshape(1, D),
                      batch_ids.astype(jnp.int32).reshape(N // BN, 1, BN))
    return out


# speedup vs baseline: 22.0133x; 1.4811x over previous
import dataclasses

import jax
import jax.numpy as jnp
from jax import lax
from jax.experimental import pallas as pl
from jax.experimental.pallas import tpu as pltpu
from jax.experimental.pallas import tpu_sc as plsc

N = 10000
E = 320000
D = 128
R = 18
RP = R + 1
B = 128

NSC = 2
NSUB = 16
NW = NSC * NSUB
EPT = E // NW
KB = 80
NBLK = EPT // KB
CPP = 180224
CPT = CPP // NW
ZR = 80

_mesh = plsc.VectorSubcoreMesh(core_axis_name="c", subcore_axis_name="s")

_sc_params = pltpu.CompilerParams()
if "needs_layout_passes" in pltpu.CompilerParams.__dataclass_fields__:
    _sc_params = dataclasses.replace(_sc_params, needs_layout_passes=False)


def _zero_buf_rows(buf, rows):
    @pl.loop(0, rows)
    def _(i):
        for j in range(D // 16):
            buf[i, pl.ds(j * 16, 16)] = jnp.zeros((16,), jnp.float32)


def _gather_x(node_embeddings, entity_ids):
    @pl.kernel(
        out_type=jax.ShapeDtypeStruct((N, D), jnp.float32),
        mesh=_mesh,
        scratch_types=[
            pltpu.VMEM((KB,), jnp.int32),
            pltpu.VMEM((KB, D), jnp.float32),
            pltpu.SemaphoreType.DMA,
        ],
    )
    def k(tab_hbm, eid_hbm, x_hbm, idx_v, rows_v, sem):
        w = lax.axis_index("c") * NSUB + lax.axis_index("s")

        @pl.loop(w, N // KB, step=NW)
        def _(b):
            off = b * KB
            pltpu.sync_copy(eid_hbm.at[pl.ds(off, KB)], idx_v)
            pltpu.async_copy(tab_hbm.at[idx_v], rows_v, sem).wait()
            pltpu.sync_copy(rows_v, x_hbm.at[pl.ds(off, KB)])

    return k(node_embeddings, entity_ids)


def _edge_counts(dst, typ):
    @pl.kernel(
        out_type=jax.ShapeDtypeStruct((NSC, CPP), jnp.float32),
        mesh=_mesh,
        scratch_types=[
            pltpu.VMEM((KB,), jnp.int32),
            pltpu.VMEM((KB,), jnp.int32),
            pltpu.VMEM((KB,), jnp.int32),
            pltpu.VMEM((KB,), jnp.float32),
            pltpu.VMEM((CPT,), jnp.float32),
            pltpu.VMEM_SHARED((CPP,), jnp.float32),
        ],
    )
    def k(dst_hbm, typ_hbm, out_hbm, dst_v, typ_v, key_v, ones_v, z_v, acc_sh):
        c = lax.axis_index("c")
        s = lax.axis_index("s")
        w = c * NSUB + s

        for j in range(KB // 16):
            ones_v[pl.ds(j * 16, 16)] = jnp.ones((16,), jnp.float32)

        @pl.loop(0, CPT, step=16)
        def _(i):
            z_v[pl.ds(i, 16)] = jnp.zeros((16,), jnp.float32)

        pltpu.sync_copy(z_v, acc_sh.at[pl.ds(s * CPT * 2, CPT)])
        pltpu.sync_copy(z_v, acc_sh.at[pl.ds(s * CPT * 2 + CPT, CPT)])
        plsc.subcore_barrier()

        base = w * EPT

        @pl.loop(0, NBLK)
        def _(i):
            off = base + i * KB
            pltpu.sync_copy(dst_hbm.at[pl.ds(off, KB)], dst_v)
            pltpu.sync_copy(typ_hbm.at[pl.ds(off, KB)], typ_v)
            for j in range(KB // 16):
                sl = pl.ds(j * 16, 16)
                key_v[sl] = dst_v[sl] * R + typ_v[sl]
            pltpu.sync_copy(ones_v, acc_sh.at[key_v], add=True)

        plsc.subcore_barrier()
        pltpu.sync_copy(acc_sh.at[pl.ds(s * CPT * 2, CPT * 2)],
                        out_hbm.at[c, pl.ds(s * CPT * 2, CPT * 2)])

    return k(dst, typ)


def _inv_counts(cnt):
    @pl.kernel(
        out_type=jax.ShapeDtypeStruct((CPP,), jnp.float32),
        mesh=_mesh,
        scratch_types=[
            pltpu.VMEM((CPT,), jnp.float32),
            pltpu.VMEM((CPT,), jnp.float32),
        ],
    )
    def k(cnt_hbm, out_hbm, a_v, b_v):
        w = lax.axis_index("c") * NSUB + lax.axis_index("s")
        base = w * CPT
        pltpu.sync_copy(cnt_hbm.at[0, pl.ds(base, CPT)], a_v)
        pltpu.sync_copy(cnt_hbm.at[1, pl.ds(base, CPT)], b_v)

        @pl.loop(0, CPT, step=16)
        def _(i):
            sl = pl.ds(i, 16)
            a_v[sl] = 1.0 / jnp.maximum(a_v[sl] + b_v[sl], 1.0)

        pltpu.sync_copy(a_v, out_hbm.at[pl.ds(base, CPT)])

    return k(cnt)


def _edge_aggregate(pk, inv, yflat):
    @pl.kernel(
        out_type=jax.ShapeDtypeStruct((NSC, N, D), jnp.float32),
        mesh=_mesh,
        compiler_params=_sc_params,
        scratch_types=[
            pltpu.VMEM((3, KB), jnp.int32),
            pltpu.VMEM((3, KB), jnp.int32),
            pltpu.VMEM((KB,), jnp.int32),
            pltpu.VMEM((KB,), jnp.int32),
            pltpu.VMEM((KB,), jnp.int32),
            pltpu.VMEM((KB,), jnp.int32),
            pltpu.VMEM((KB + 16,), jnp.float32),
            pltpu.VMEM((KB + 16,), jnp.float32),
            pltpu.VMEM((KB, D), jnp.float32),
            pltpu.VMEM((KB, D), jnp.float32),
            pltpu.VMEM((ZR, D), jnp.float32),
            pltpu.VMEM_SHARED((N, D), jnp.float32),
            pltpu.SemaphoreType.DMA,
            pltpu.SemaphoreType.DMA,
            pltpu.SemaphoreType.DMA,
            pltpu.SemaphoreType.DMA,
        ],
    )
    def k(pk_hbm, inv_hbm, y_hbm, out_hbm,
          pk0, pk1, k1a, k1b, k2a, k2b, wa, wb, ra, rb, z_v, acc_sh,
          semr0, semr1, semi0, semi1):
        c = lax.axis_index("c")
        s = lax.axis_index("s")
        w = c * NSUB + s

        _zero_buf_rows(z_v, ZR)

        @pl.loop(s, N // ZR, step=NSUB)
        def _(g):
            pltpu.sync_copy(z_v, acc_sh.at[pl.ds(g * ZR, ZR)])

        plsc.subcore_barrier()

        bufs = ((pk0, k1a, k2a, wa, ra, semr0, semi0),
                (pk1, k1b, k2b, wb, rb, semr1, semi1))
        base = w * NBLK

        def load_and_fire(b, t):
            pk_v, k1_v, k2_v, w_v, rows_v, semr, semi = bufs[t]
            pltpu.sync_copy(pk_hbm.at[b], pk_v)
            for j in range(KB // 16):
                sl = pl.ds(j * 16, 16)
                t16 = pk_v[2, sl]
                k1_v[sl] = pk_v[0, sl] * RP + t16
                k2_v[sl] = pk_v[1, sl] * R + t16
            pltpu.async_copy(y_hbm.at[k1_v], rows_v, semr)
            pltpu.async_copy(inv_hbm.at[k2_v], w_v.at[pl.ds(16, KB)], semi)

        def drain_and_scatter(t):
            pk_v, k1_v, k2_v, w_v, rows_v, semr, semi = bufs[t]
            pltpu.make_async_copy(y_hbm.at[k1_v], rows_v, semr).wait()
            pltpu.make_async_copy(inv_hbm.at[k2_v], w_v.at[pl.ds(16, KB)],
                                  semi).wait()
            @plsc.parallel_loop(0, KB, unroll=8)
            def _(e):
                wv = plsc.load_gather(
                    w_v, [jnp.full((16,), e + 16, jnp.int32)])
                for j in range(D // 16):
                    sl = pl.ds(j * 16, 16)
                    rows_v[e, sl] = rows_v[e, sl] * wv

            pltpu.sync_copy(rows_v, acc_sh.at[pk_v.at[1]], add=True)

        load_and_fire(base, 0)
        load_and_fire(base + 1, 1)

        @pl.loop(0, (NBLK - 1) // 2)
        def _(p):
            drain_and_scatter(0)
            load_and_fire(base + 2 * p + 2, 0)
            drain_and_scatter(1)

            @pl.when(2 * p + 3 < NBLK)
            def _():
                load_and_fire(base + 2 * p + 3, 1)

        drain_and_scatter(0)

        plsc.subcore_barrier()

        @pl.loop(s, N // ZR, step=NSUB)
        def _(g):
            pltpu.sync_copy(acc_sh.at[pl.ds(g * ZR, ZR)],
                            out_hbm.at[c, pl.ds(g * ZR, ZR)])

    return k(pk, inv, yflat)


BN = 400


def _y_from_x(x, wall):
    def body(x_ref, w_ref, o_ref):
        o_ref[...] = jnp.dot(
            x_ref[...], w_ref[...],
            preferred_element_type=jnp.float32).reshape(BN, RP, D)

    return pl.pallas_call(
        body,
        grid=(N // BN,),
        in_specs=[
            pl.BlockSpec((BN, D), lambda i: (i, 0)),
            pl.BlockSpec((D, RP * D), lambda i: (0, 0)),
        ],
        out_specs=pl.BlockSpec((BN, RP, D), lambda i: (i, 0, 0)),
        out_shape=jax.ShapeDtypeStruct((N, RP, D), jnp.float32),
    )(x, wall)


def _y_from_agg(agg0, agg1, yprev, bias, wall):
    def body(a0_ref, a1_ref, yr_ref, b_ref, w_ref, o_ref):
        h = a0_ref[...] + a1_ref[...] + yr_ref[...] + b_ref[0][None, :]
        o_ref[...] = jnp.dot(
            h, w_ref[...],
            preferred_element_type=jnp.float32).reshape(BN, RP, D)

    return pl.pallas_call(
        body,
        grid=(N // BN,),
        in_specs=[
            pl.BlockSpec((BN, D), lambda i: (i, 0)),
            pl.BlockSpec((BN, D), lambda i: (i, 0)),
            pl.BlockSpec((BN, D), lambda i: (i, 0)),
            pl.BlockSpec((1, D), lambda i: (0, 0)),
            pl.BlockSpec((D, RP * D), lambda i: (0, 0)),
        ],
        out_specs=pl.BlockSpec((BN, RP, D), lambda i: (i, 0, 0)),
        out_shape=jax.ShapeDtypeStruct((N, RP, D), jnp.float32),
    )(agg0, agg1, yprev, bias, wall)


def _batch_pool(agg0, agg1, yprev, bias, bid3):
    def body(a0_ref, a1_ref, yr_ref, b_ref, bid_ref, o_ref):
        h = a0_ref[...] + a1_ref[...] + yr_ref[...] + b_ref[0][None, :]
        bb = bid_ref[0, 0, :]
        oh = (lax.broadcasted_iota(jnp.int32, (B, BN), 0)
              == bb[None, :]).astype(jnp.float32)

        @pl.when(pl.program_id(0) == 0)
        def _():
            o_ref[...] = jnp.zeros_like(o_ref)

        o_ref[...] += jnp.dot(oh, h, preferred_element_type=jnp.float32)

    return pl.pallas_call(
        body,
        grid=(N // BN,),
        in_specs=[
            pl.BlockSpec((BN, D), lambda i: (i, 0)),
            pl.BlockSpec((BN, D), lambda i: (i, 0)),
            pl.BlockSpec((BN, D), lambda i: (i, 0)),
            pl.BlockSpec((1, D), lambda i: (0, 0)),
            pl.BlockSpec((1, 1, BN), lambda i: (i, 0, 0)),
        ],
        out_specs=pl.BlockSpec((B, D), lambda i: (0, 0)),
        out_shape=jax.ShapeDtypeStruct((B, D), jnp.float32),
    )(agg0, agg1, yprev, bias, bid3)


def kernel(edge_index, edge_type, entity_ids, batch_ids, node_embeddings,
           W1, root1, b1, W2, root2, b2):
    src = edge_index[0]
    dst = edge_index[1]
    typ = edge_type.astype(jnp.int32)
    pk = jnp.stack([src, dst, typ]).reshape(3, E // KB, KB).transpose(1, 0, 2)

    x = _gather_x(node_embeddings, entity_ids.astype(jnp.int32))
    cnt = _edge_counts(dst, typ)
    inv = _inv_counts(cnt)

    w1a = jnp.concatenate([W1, root1[None]], axis=0)
    w2a = jnp.concatenate([W2, root2[None]], axis=0)
    w1f = w1a.transpose(1, 0, 2).reshape(D, RP * D)
    w2f = w2a.transpose(1, 0, 2).reshape(D, RP * D)

    y1 = _y_from_x(x, w1f)
    agg1 = _edge_aggregate(pk, inv, y1.reshape(N * RP, D))
    y2 = _y_from_agg(agg1[0], agg1[1], y1[:, R, :], b1.reshape(1, D), w2f)
    agg2 = _edge_aggregate(pk, inv, y2.reshape(N * RP, D))
    out = _batch_pool(agg2[0], agg2[1], y2[:, R, :], b2.reshape(1, D),
                      batch_ids.astype(jnp.int32).reshape(N // BN, 1, BN))
    return out


# speedup vs baseline: 22.1150x; 1.0046x over previous
import dataclasses

import jax
import jax.numpy as jnp
from jax import lax
from jax.experimental import pallas as pl
from jax.experimental.pallas import tpu as pltpu
from jax.experimental.pallas import tpu_sc as plsc

N = 10000
E = 320000
D = 128
R = 18
RP = R + 1
B = 128

NSC = 2
NSUB = 16
NW = NSC * NSUB
EPT = E // NW
KB = 80
NBLK = EPT // KB
CPP = 180224
CPT = CPP // NW
ZR = 80

_mesh = plsc.VectorSubcoreMesh(core_axis_name="c", subcore_axis_name="s")

_sc_params = pltpu.CompilerParams()
if "needs_layout_passes" in pltpu.CompilerParams.__dataclass_fields__:
    _sc_params = dataclasses.replace(_sc_params, needs_layout_passes=False)


def _zero_buf_rows(buf, rows):
    @pl.loop(0, rows)
    def _(i):
        for j in range(D // 16):
            buf[i, pl.ds(j * 16, 16)] = jnp.zeros((16,), jnp.float32)


def _gather_x(node_embeddings, entity_ids):
    @pl.kernel(
        out_type=jax.ShapeDtypeStruct((N, D), jnp.float32),
        mesh=_mesh,
        scratch_types=[
            pltpu.VMEM((KB,), jnp.int32),
            pltpu.VMEM((KB, D), jnp.float32),
            pltpu.SemaphoreType.DMA,
        ],
    )
    def k(tab_hbm, eid_hbm, x_hbm, idx_v, rows_v, sem):
        w = lax.axis_index("c") * NSUB + lax.axis_index("s")

        @pl.loop(w, N // KB, step=NW)
        def _(b):
            off = b * KB
            pltpu.sync_copy(eid_hbm.at[pl.ds(off, KB)], idx_v)
            pltpu.async_copy(tab_hbm.at[idx_v], rows_v, sem).wait()
            pltpu.sync_copy(rows_v, x_hbm.at[pl.ds(off, KB)])

    return k(node_embeddings, entity_ids)


def _edge_counts(dst, typ):
    @pl.kernel(
        out_type=jax.ShapeDtypeStruct((NSC, CPP), jnp.float32),
        mesh=_mesh,
        scratch_types=[
            pltpu.VMEM((KB,), jnp.int32),
            pltpu.VMEM((KB,), jnp.int32),
            pltpu.VMEM((KB,), jnp.int32),
            pltpu.VMEM((KB,), jnp.float32),
            pltpu.VMEM((CPT,), jnp.float32),
            pltpu.VMEM_SHARED((CPP,), jnp.float32),
        ],
    )
    def k(dst_hbm, typ_hbm, out_hbm, dst_v, typ_v, key_v, ones_v, z_v, acc_sh):
        c = lax.axis_index("c")
        s = lax.axis_index("s")
        w = c * NSUB + s

        for j in range(KB // 16):
            ones_v[pl.ds(j * 16, 16)] = jnp.ones((16,), jnp.float32)

        @pl.loop(0, CPT, step=16)
        def _(i):
            z_v[pl.ds(i, 16)] = jnp.zeros((16,), jnp.float32)

        pltpu.sync_copy(z_v, acc_sh.at[pl.ds(s * CPT * 2, CPT)])
        pltpu.sync_copy(z_v, acc_sh.at[pl.ds(s * CPT * 2 + CPT, CPT)])
        plsc.subcore_barrier()

        base = w * EPT

        @pl.loop(0, NBLK)
        def _(i):
            off = base + i * KB
            pltpu.sync_copy(dst_hbm.at[pl.ds(off, KB)], dst_v)
            pltpu.sync_copy(typ_hbm.at[pl.ds(off, KB)], typ_v)
            for j in range(KB // 16):
                sl = pl.ds(j * 16, 16)
                key_v[sl] = dst_v[sl] * R + typ_v[sl]
            pltpu.sync_copy(ones_v, acc_sh.at[key_v], add=True)

        plsc.subcore_barrier()
        pltpu.sync_copy(acc_sh.at[pl.ds(s * CPT * 2, CPT * 2)],
                        out_hbm.at[c, pl.ds(s * CPT * 2, CPT * 2)])

    return k(dst, typ)


def _inv_counts(cnt):
    @pl.kernel(
        out_type=jax.ShapeDtypeStruct((CPP,), jnp.float32),
        mesh=_mesh,
        scratch_types=[
            pltpu.VMEM((CPT,), jnp.float32),
            pltpu.VMEM((CPT,), jnp.float32),
        ],
    )
    def k(cnt_hbm, out_hbm, a_v, b_v):
        w = lax.axis_index("c") * NSUB + lax.axis_index("s")
        base = w * CPT
        pltpu.sync_copy(cnt_hbm.at[0, pl.ds(base, CPT)], a_v)
        pltpu.sync_copy(cnt_hbm.at[1, pl.ds(base, CPT)], b_v)

        @pl.loop(0, CPT, step=16)
        def _(i):
            sl = pl.ds(i, 16)
            a_v[sl] = 1.0 / jnp.maximum(a_v[sl] + b_v[sl], 1.0)

        pltpu.sync_copy(a_v, out_hbm.at[pl.ds(base, CPT)])

    return k(cnt)


def _edge_aggregate(pk, inv, yflat):
    @pl.kernel(
        out_type=jax.ShapeDtypeStruct((NSC, N, D), jnp.float32),
        mesh=_mesh,
        compiler_params=_sc_params,
        scratch_types=[
            pltpu.VMEM((3, KB), jnp.int32),
            pltpu.VMEM((3, KB), jnp.int32),
            pltpu.VMEM((KB,), jnp.int32),
            pltpu.VMEM((KB,), jnp.int32),
            pltpu.VMEM((KB,), jnp.int32),
            pltpu.VMEM((KB,), jnp.int32),
            pltpu.VMEM((KB + 16,), jnp.float32),
            pltpu.VMEM((KB + 16,), jnp.float32),
            pltpu.VMEM((KB, D), jnp.float32),
            pltpu.VMEM((KB, D), jnp.float32),
            pltpu.VMEM((ZR, D), jnp.float32),
            pltpu.VMEM_SHARED((N, D), jnp.float32),
            pltpu.SemaphoreType.DMA,
            pltpu.SemaphoreType.DMA,
            pltpu.SemaphoreType.DMA,
            pltpu.SemaphoreType.DMA,
        ],
    )
    def k(pk_hbm, inv_hbm, y_hbm, out_hbm,
          pk0, pk1, k1a, k1b, k2a, k2b, wa, wb, ra, rb, z_v, acc_sh,
          semr0, semr1, semi0, semi1):
        c = lax.axis_index("c")
        s = lax.axis_index("s")
        w = c * NSUB + s

        _zero_buf_rows(z_v, ZR)

        @pl.loop(s, N // ZR, step=NSUB)
        def _(g):
            pltpu.sync_copy(z_v, acc_sh.at[pl.ds(g * ZR, ZR)])

        plsc.subcore_barrier()

        bufs = ((pk0, k1a, k2a, wa, ra, semr0, semi0),
                (pk1, k1b, k2b, wb, rb, semr1, semi1))
        base = w * NBLK

        def load_and_fire(b, t):
            pk_v, k1_v, k2_v, w_v, rows_v, semr, semi = bufs[t]
            pltpu.sync_copy(pk_hbm.at[b], pk_v)
            for j in range(KB // 16):
                sl = pl.ds(j * 16, 16)
                t16 = pk_v[2, sl]
                k1_v[sl] = pk_v[0, sl] * RP + t16
                k2_v[sl] = pk_v[1, sl] * R + t16
            pltpu.async_copy(y_hbm.at[k1_v], rows_v, semr)
            pltpu.async_copy(inv_hbm.at[k2_v], w_v.at[pl.ds(16, KB)], semi)

        def drain_and_scatter(t):
            pk_v, k1_v, k2_v, w_v, rows_v, semr, semi = bufs[t]
            pltpu.make_async_copy(y_hbm.at[k1_v], rows_v, semr).wait()
            pltpu.make_async_copy(inv_hbm.at[k2_v], w_v.at[pl.ds(16, KB)],
                                  semi).wait()
            @plsc.parallel_loop(0, KB, unroll=8)
            def _(e):
                wv = plsc.load_gather(
                    w_v, [jnp.full((16,), e + 16, jnp.int32)])
                for j in range(D // 16):
                    sl = pl.ds(j * 16, 16)
                    rows_v[e, sl] = rows_v[e, sl] * wv

            pltpu.sync_copy(rows_v, acc_sh.at[pk_v.at[1]], add=True)

        load_and_fire(base, 0)
        load_and_fire(base + 1, 1)

        @pl.loop(0, (NBLK - 1) // 2)
        def _(p):
            drain_and_scatter(0)
            load_and_fire(base + 2 * p + 2, 0)
            drain_and_scatter(1)

            @pl.when(2 * p + 3 < NBLK)
            def _():
                load_and_fire(base + 2 * p + 3, 1)

        drain_and_scatter(0)

        plsc.subcore_barrier()

        @pl.loop(s, N // ZR, step=NSUB)
        def _(g):
            pltpu.sync_copy(acc_sh.at[pl.ds(g * ZR, ZR)],
                            out_hbm.at[c, pl.ds(g * ZR, ZR)])

    return k(pk, inv, yflat)


BN = 400


def _y_from_x(x, wall):
    def body(x_ref, w_ref, o_ref):
        o_ref[...] = jnp.dot(
            x_ref[...].astype(jnp.bfloat16), w_ref[...],
            preferred_element_type=jnp.float32).reshape(BN, RP, D)

    return pl.pallas_call(
        body,
        grid=(N // BN,),
        in_specs=[
            pl.BlockSpec((BN, D), lambda i: (i, 0)),
            pl.BlockSpec((D, RP * D), lambda i: (0, 0)),
        ],
        out_specs=pl.BlockSpec((BN, RP, D), lambda i: (i, 0, 0)),
        out_shape=jax.ShapeDtypeStruct((N, RP, D), jnp.float32),
    )(x, wall)


def _y_from_agg(agg0, agg1, yprev, bias, wall):
    def body(a0_ref, a1_ref, yr_ref, b_ref, w_ref, o_ref):
        h = a0_ref[...] + a1_ref[...] + yr_ref[...] + b_ref[0][None, :]
        o_ref[...] = jnp.dot(
            h.astype(jnp.bfloat16), w_ref[...],
            preferred_element_type=jnp.float32).reshape(BN, RP, D)

    return pl.pallas_call(
        body,
        grid=(N // BN,),
        in_specs=[
            pl.BlockSpec((BN, D), lambda i: (i, 0)),
            pl.BlockSpec((BN, D), lambda i: (i, 0)),
            pl.BlockSpec((BN, D), lambda i: (i, 0)),
            pl.BlockSpec((1, D), lambda i: (0, 0)),
            pl.BlockSpec((D, RP * D), lambda i: (0, 0)),
        ],
        out_specs=pl.BlockSpec((BN, RP, D), lambda i: (i, 0, 0)),
        out_shape=jax.ShapeDtypeStruct((N, RP, D), jnp.float32),
    )(agg0, agg1, yprev, bias, wall)


def _batch_pool(agg0, agg1, yprev, bias, bid3):
    def body(a0_ref, a1_ref, yr_ref, b_ref, bid_ref, o_ref):
        h = a0_ref[...] + a1_ref[...] + yr_ref[...] + b_ref[0][None, :]
        bb = bid_ref[0, 0, :]
        oh = (lax.broadcasted_iota(jnp.int32, (B, BN), 0)
              == bb[None, :]).astype(jnp.float32)

        @pl.when(pl.program_id(0) == 0)
        def _():
            o_ref[...] = jnp.zeros_like(o_ref)

        o_ref[...] += jnp.dot(oh, h, preferred_element_type=jnp.float32)

    return pl.pallas_call(
        body,
        grid=(N // BN,),
        in_specs=[
            pl.BlockSpec((BN, D), lambda i: (i, 0)),
            pl.BlockSpec((BN, D), lambda i: (i, 0)),
            pl.BlockSpec((BN, D), lambda i: (i, 0)),
            pl.BlockSpec((1, D), lambda i: (0, 0)),
            pl.BlockSpec((1, 1, BN), lambda i: (i, 0, 0)),
        ],
        out_specs=pl.BlockSpec((B, D), lambda i: (0, 0)),
        out_shape=jax.ShapeDtypeStruct((B, D), jnp.float32),
    )(agg0, agg1, yprev, bias, bid3)


def kernel(edge_index, edge_type, entity_ids, batch_ids, node_embeddings,
           W1, root1, b1, W2, root2, b2):
    src = edge_index[0]
    dst = edge_index[1]
    typ = edge_type.astype(jnp.int32)
    pk = jnp.stack([src, dst, typ]).reshape(3, E // KB, KB).transpose(1, 0, 2)

    x = _gather_x(node_embeddings, entity_ids.astype(jnp.int32))
    cnt = _edge_counts(dst, typ)
    inv = _inv_counts(cnt)

    w1a = jnp.concatenate([W1, root1[None]], axis=0)
    w2a = jnp.concatenate([W2, root2[None]], axis=0)
    w1f = w1a.transpose(1, 0, 2).reshape(D, RP * D).astype(jnp.bfloat16)
    w2f = w2a.transpose(1, 0, 2).reshape(D, RP * D).astype(jnp.bfloat16)

    y1 = _y_from_x(x, w1f)
    agg1 = _edge_aggregate(pk, inv, y1.reshape(N * RP, D))
    y2 = _y_from_agg(agg1[0], agg1[1], y1[:, R, :], b1.reshape(1, D), w2f)
    agg2 = _edge_aggregate(pk, inv, y2.reshape(N * RP, D))
    out = _batch_pool(agg2[0], agg2[1], y2[:, R, :], b2.reshape(1, D),
                      batch_ids.astype(jnp.int32).reshape(N // BN, 1, BN))
    return out


# speedup vs baseline: 29.5483x; 1.3361x over previous
import dataclasses

import jax
import jax.numpy as jnp
from jax import lax
from jax.experimental import pallas as pl
from jax.experimental.pallas import tpu as pltpu
from jax.experimental.pallas import tpu_sc as plsc

N = 10000
E = 320000
D = 128
R = 18
RP = R + 1
B = 128

NSC = 2
NSUB = 16
NW = NSC * NSUB
EPT = E // NW
KB = 80
NBLK = EPT // KB
CPP = 180224
CPT = CPP // NW
ZR = 80

_mesh = plsc.VectorSubcoreMesh(core_axis_name="c", subcore_axis_name="s")

_sc_params = pltpu.CompilerParams()
if "needs_layout_passes" in pltpu.CompilerParams.__dataclass_fields__:
    _sc_params = dataclasses.replace(_sc_params, needs_layout_passes=False)


def _zero_buf_rows(buf, rows):
    @pl.loop(0, rows)
    def _(i):
        for j in range(D // 16):
            buf[i, pl.ds(j * 16, 16)] = jnp.zeros((16,), jnp.float32)


def _gather_x(node_embeddings, entity_ids):
    @pl.kernel(
        out_type=jax.ShapeDtypeStruct((N, D), jnp.float32),
        mesh=_mesh,
        scratch_types=[
            pltpu.VMEM((KB,), jnp.int32),
            pltpu.VMEM((KB, D), jnp.float32),
            pltpu.SemaphoreType.DMA,
        ],
    )
    def k(tab_hbm, eid_hbm, x_hbm, idx_v, rows_v, sem):
        w = lax.axis_index("c") * NSUB + lax.axis_index("s")

        @pl.loop(w, N // KB, step=NW)
        def _(b):
            off = b * KB
            pltpu.sync_copy(eid_hbm.at[pl.ds(off, KB)], idx_v)
            pltpu.async_copy(tab_hbm.at[idx_v], rows_v, sem).wait()
            pltpu.sync_copy(rows_v, x_hbm.at[pl.ds(off, KB)])

    return k(node_embeddings, entity_ids)


def _edge_counts(dst, typ):
    @pl.kernel(
        out_type=jax.ShapeDtypeStruct((NSC, CPP), jnp.float32),
        mesh=_mesh,
        scratch_types=[
            pltpu.VMEM((KB,), jnp.int32),
            pltpu.VMEM((KB,), jnp.int32),
            pltpu.VMEM((KB,), jnp.int32),
            pltpu.VMEM((KB,), jnp.float32),
            pltpu.VMEM((CPT,), jnp.float32),
            pltpu.VMEM_SHARED((CPP,), jnp.float32),
        ],
    )
    def k(dst_hbm, typ_hbm, out_hbm, dst_v, typ_v, key_v, ones_v, z_v, acc_sh):
        c = lax.axis_index("c")
        s = lax.axis_index("s")
        w = c * NSUB + s

        for j in range(KB // 16):
            ones_v[pl.ds(j * 16, 16)] = jnp.ones((16,), jnp.float32)

        @pl.loop(0, CPT, step=16)
        def _(i):
            z_v[pl.ds(i, 16)] = jnp.zeros((16,), jnp.float32)

        pltpu.sync_copy(z_v, acc_sh.at[pl.ds(s * CPT * 2, CPT)])
        pltpu.sync_copy(z_v, acc_sh.at[pl.ds(s * CPT * 2 + CPT, CPT)])
        plsc.subcore_barrier()

        base = w * EPT

        @pl.loop(0, NBLK)
        def _(i):
            off = base + i * KB
            pltpu.sync_copy(dst_hbm.at[pl.ds(off, KB)], dst_v)
            pltpu.sync_copy(typ_hbm.at[pl.ds(off, KB)], typ_v)
            for j in range(KB // 16):
                sl = pl.ds(j * 16, 16)
                key_v[sl] = dst_v[sl] * R + typ_v[sl]
            pltpu.sync_copy(ones_v, acc_sh.at[key_v], add=True)

        plsc.subcore_barrier()
        pltpu.sync_copy(acc_sh.at[pl.ds(s * CPT * 2, CPT * 2)],
                        out_hbm.at[c, pl.ds(s * CPT * 2, CPT * 2)])

    return k(dst, typ)


def _inv_counts(cnt):
    @pl.kernel(
        out_type=jax.ShapeDtypeStruct((CPP,), jnp.float32),
        mesh=_mesh,
        scratch_types=[
            pltpu.VMEM((CPT,), jnp.float32),
            pltpu.VMEM((CPT,), jnp.float32),
        ],
    )
    def k(cnt_hbm, out_hbm, a_v, b_v):
        w = lax.axis_index("c") * NSUB + lax.axis_index("s")
        base = w * CPT
        pltpu.sync_copy(cnt_hbm.at[0, pl.ds(base, CPT)], a_v)
        pltpu.sync_copy(cnt_hbm.at[1, pl.ds(base, CPT)], b_v)

        @pl.loop(0, CPT, step=16)
        def _(i):
            sl = pl.ds(i, 16)
            a_v[sl] = 1.0 / jnp.maximum(a_v[sl] + b_v[sl], 1.0)

        pltpu.sync_copy(a_v, out_hbm.at[pl.ds(base, CPT)])

    return k(cnt)


def _edge_aggregate(pk, inv, yflat):
    @pl.kernel(
        out_type=jax.ShapeDtypeStruct((NSC, N, D), jnp.float32),
        mesh=_mesh,
        compiler_params=_sc_params,
        scratch_types=[
            pltpu.VMEM((3, KB), jnp.int32),
            pltpu.VMEM((3, KB), jnp.int32),
            pltpu.VMEM((KB,), jnp.int32),
            pltpu.VMEM((KB,), jnp.int32),
            pltpu.VMEM((KB,), jnp.int32),
            pltpu.VMEM((KB,), jnp.int32),
            pltpu.VMEM((KB + 16,), jnp.float32),
            pltpu.VMEM((KB + 16,), jnp.float32),
            pltpu.VMEM((KB, D), jnp.float32),
            pltpu.VMEM((KB, D), jnp.float32),
            pltpu.VMEM((ZR, D), jnp.float32),
            pltpu.VMEM_SHARED((N, D), jnp.float32),
            pltpu.SemaphoreType.DMA,
            pltpu.SemaphoreType.DMA,
            pltpu.SemaphoreType.DMA,
            pltpu.SemaphoreType.DMA,
        ],
    )
    def k(pk_hbm, inv_hbm, y_hbm, out_hbm,
          pk0, pk1, k1a, k1b, k2a, k2b, wa, wb, ra, rb, z_v, acc_sh,
          semr0, semr1, semi0, semi1):
        c = lax.axis_index("c")
        s = lax.axis_index("s")
        w = c * NSUB + s

        _zero_buf_rows(z_v, ZR)

        @pl.loop(s, N // ZR, step=NSUB)
        def _(g):
            pltpu.sync_copy(z_v, acc_sh.at[pl.ds(g * ZR, ZR)])

        plsc.subcore_barrier()

        bufs = ((pk0, k1a, k2a, wa, ra, semr0, semi0),
                (pk1, k1b, k2b, wb, rb, semr1, semi1))
        base = w * NBLK

        def load_and_fire(b, t):
            pk_v, k1_v, k2_v, w_v, rows_v, semr, semi = bufs[t]
            pltpu.sync_copy(pk_hbm.at[b], pk_v)
            for j in range(KB // 16):
                sl = pl.ds(j * 16, 16)
                t16 = pk_v[2, sl]
                k1_v[sl] = t16 * N + pk_v[0, sl]
                k2_v[sl] = pk_v[1, sl] * R + t16
            pltpu.async_copy(y_hbm.at[k1_v], rows_v, semr)
            pltpu.async_copy(inv_hbm.at[k2_v], w_v.at[pl.ds(16, KB)], semi)

        def drain_and_scatter(t):
            pk_v, k1_v, k2_v, w_v, rows_v, semr, semi = bufs[t]
            pltpu.make_async_copy(y_hbm.at[k1_v], rows_v, semr).wait()
            pltpu.make_async_copy(inv_hbm.at[k2_v], w_v.at[pl.ds(16, KB)],
                                  semi).wait()
            @plsc.parallel_loop(0, KB, unroll=8)
            def _(e):
                wv = plsc.load_gather(
                    w_v, [jnp.full((16,), e + 16, jnp.int32)])
                for j in range(D // 16):
                    sl = pl.ds(j * 16, 16)
                    rows_v[e, sl] = rows_v[e, sl] * wv

            pltpu.sync_copy(rows_v, acc_sh.at[pk_v.at[1]], add=True)

        load_and_fire(base, 0)
        load_and_fire(base + 1, 1)

        @pl.loop(0, (NBLK - 1) // 2)
        def _(p):
            drain_and_scatter(0)
            load_and_fire(base + 2 * p + 2, 0)
            drain_and_scatter(1)

            @pl.when(2 * p + 3 < NBLK)
            def _():
                load_and_fire(base + 2 * p + 3, 1)

        drain_and_scatter(0)

        plsc.subcore_barrier()

        @pl.loop(s, N // ZR, step=NSUB)
        def _(g):
            pltpu.sync_copy(acc_sh.at[pl.ds(g * ZR, ZR)],
                            out_hbm.at[c, pl.ds(g * ZR, ZR)])

    return k(pk, inv, yflat)


BN = 400


def _y_from_x(x, wall):
    def body(x_ref, w_ref, o_ref):
        res = jnp.dot(x_ref[...].astype(jnp.bfloat16), w_ref[...],
                      preferred_element_type=jnp.float32)
        for r in range(RP):
            o_ref[r] = res[:, r * D:(r + 1) * D]

    return pl.pallas_call(
        body,
        grid=(N // BN,),
        in_specs=[
            pl.BlockSpec((BN, D), lambda i: (i, 0)),
            pl.BlockSpec((D, RP * D), lambda i: (0, 0)),
        ],
        out_specs=pl.BlockSpec((RP, BN, D), lambda i: (0, i, 0)),
        out_shape=jax.ShapeDtypeStruct((RP, N, D), jnp.float32),
    )(x, wall)


def _y_from_agg(agg0, agg1, yprev, bias, wall):
    def body(a0_ref, a1_ref, yr_ref, b_ref, w_ref, o_ref):
        h = a0_ref[...] + a1_ref[...] + yr_ref[...] + b_ref[0][None, :]
        res = jnp.dot(h.astype(jnp.bfloat16), w_ref[...],
                      preferred_element_type=jnp.float32)
        for r in range(RP):
            o_ref[r] = res[:, r * D:(r + 1) * D]

    return pl.pallas_call(
        body,
        grid=(N // BN,),
        in_specs=[
            pl.BlockSpec((BN, D), lambda i: (i, 0)),
            pl.BlockSpec((BN, D), lambda i: (i, 0)),
            pl.BlockSpec((BN, D), lambda i: (i, 0)),
            pl.BlockSpec((1, D), lambda i: (0, 0)),
            pl.BlockSpec((D, RP * D), lambda i: (0, 0)),
        ],
        out_specs=pl.BlockSpec((RP, BN, D), lambda i: (0, i, 0)),
        out_shape=jax.ShapeDtypeStruct((RP, N, D), jnp.float32),
    )(agg0, agg1, yprev, bias, wall)


def _batch_pool(agg0, agg1, yprev, bias, bid3):
    def body(a0_ref, a1_ref, yr_ref, b_ref, bid_ref, o_ref):
        h = a0_ref[...] + a1_ref[...] + yr_ref[...] + b_ref[0][None, :]
        bb = bid_ref[0, 0, :]
        oh = (lax.broadcasted_iota(jnp.int32, (B, BN), 0)
              == bb[None, :]).astype(jnp.float32)

        @pl.when(pl.program_id(0) == 0)
        def _():
            o_ref[...] = jnp.zeros_like(o_ref)

        o_ref[...] += jnp.dot(oh, h, preferred_element_type=jnp.float32)

    return pl.pallas_call(
        body,
        grid=(N // BN,),
        in_specs=[
            pl.BlockSpec((BN, D), lambda i: (i, 0)),
            pl.BlockSpec((BN, D), lambda i: (i, 0)),
            pl.BlockSpec((BN, D), lambda i: (i, 0)),
            pl.BlockSpec((1, D), lambda i: (0, 0)),
            pl.BlockSpec((1, 1, BN), lambda i: (i, 0, 0)),
        ],
        out_specs=pl.BlockSpec((B, D), lambda i: (0, 0)),
        out_shape=jax.ShapeDtypeStruct((B, D), jnp.float32),
    )(agg0, agg1, yprev, bias, bid3)


def kernel(edge_index, edge_type, entity_ids, batch_ids, node_embeddings,
           W1, root1, b1, W2, root2, b2):
    src = edge_index[0]
    dst = edge_index[1]
    typ = edge_type.astype(jnp.int32)
    pk = jnp.stack([src, dst, typ]).reshape(3, E // KB, KB).transpose(1, 0, 2)

    x = _gather_x(node_embeddings, entity_ids.astype(jnp.int32))
    cnt = _edge_counts(dst, typ)
    inv = _inv_counts(cnt)

    w1a = jnp.concatenate([W1, root1[None]], axis=0)
    w2a = jnp.concatenate([W2, root2[None]], axis=0)
    w1f = w1a.transpose(1, 0, 2).reshape(D, RP * D).astype(jnp.bfloat16)
    w2f = w2a.transpose(1, 0, 2).reshape(D, RP * D).astype(jnp.bfloat16)

    y1 = _y_from_x(x, w1f)
    agg1 = _edge_aggregate(pk, inv, y1.reshape(RP * N, D))
    y2 = _y_from_agg(agg1[0], agg1[1], y1[R], b1.reshape(1, D), w2f)
    agg2 = _edge_aggregate(pk, inv, y2.reshape(RP * N, D))
    out = _batch_pool(agg2[0], agg2[1], y2[R], b2.reshape(1, D),
                      batch_ids.astype(jnp.int32).reshape(N // BN, 1, BN))
    return out


# speedup vs baseline: 33.0936x; 1.1200x over previous
import dataclasses

import jax
import jax.numpy as jnp
from jax import lax
from jax.experimental import pallas as pl
from jax.experimental.pallas import tpu as pltpu
from jax.experimental.pallas import tpu_sc as plsc

N = 10000
E = 320000
D = 128
R = 18
RP = R + 1
B = 128

NSC = 2
NSUB = 16
NW = NSC * NSUB
EPT = E // NW
KB = 80
NBLK = EPT // KB
CPP = 180224
CPT = CPP // NW
ZR = 80

_mesh = plsc.VectorSubcoreMesh(core_axis_name="c", subcore_axis_name="s")

_sc_params = pltpu.CompilerParams()
if "needs_layout_passes" in pltpu.CompilerParams.__dataclass_fields__:
    _sc_params = dataclasses.replace(_sc_params, needs_layout_passes=False)


def _zero_buf_rows(buf, rows):
    @pl.loop(0, rows)
    def _(i):
        for j in range(D // 16):
            buf[i, pl.ds(j * 16, 16)] = jnp.zeros((16,), jnp.float32)


def _gather_x(node_embeddings, entity_ids):
    @pl.kernel(
        out_type=jax.ShapeDtypeStruct((N, D), jnp.float32),
        mesh=_mesh,
        scratch_types=[
            pltpu.VMEM((KB,), jnp.int32),
            pltpu.VMEM((KB, D), jnp.float32),
            pltpu.SemaphoreType.DMA,
        ],
    )
    def k(tab_hbm, eid_hbm, x_hbm, idx_v, rows_v, sem):
        w = lax.axis_index("c") * NSUB + lax.axis_index("s")

        @pl.loop(w, N // KB, step=NW)
        def _(b):
            off = b * KB
            pltpu.sync_copy(eid_hbm.at[pl.ds(off, KB)], idx_v)
            pltpu.async_copy(tab_hbm.at[idx_v], rows_v, sem).wait()
            pltpu.sync_copy(rows_v, x_hbm.at[pl.ds(off, KB)])

    return k(node_embeddings, entity_ids)


def _edge_counts(dst, typ):
    @pl.kernel(
        out_type=jax.ShapeDtypeStruct((NSC, CPP), jnp.float32),
        mesh=_mesh,
        scratch_types=[
            pltpu.VMEM((KB,), jnp.int32),
            pltpu.VMEM((KB,), jnp.int32),
            pltpu.VMEM((KB,), jnp.int32),
            pltpu.VMEM((KB,), jnp.float32),
            pltpu.VMEM((CPT,), jnp.float32),
            pltpu.VMEM_SHARED((CPP,), jnp.float32),
        ],
    )
    def k(dst_hbm, typ_hbm, out_hbm, dst_v, typ_v, key_v, ones_v, z_v, acc_sh):
        c = lax.axis_index("c")
        s = lax.axis_index("s")
        w = c * NSUB + s

        for j in range(KB // 16):
            ones_v[pl.ds(j * 16, 16)] = jnp.ones((16,), jnp.float32)

        @pl.loop(0, CPT, step=16)
        def _(i):
            z_v[pl.ds(i, 16)] = jnp.zeros((16,), jnp.float32)

        pltpu.sync_copy(z_v, acc_sh.at[pl.ds(s * CPT * 2, CPT)])
        pltpu.sync_copy(z_v, acc_sh.at[pl.ds(s * CPT * 2 + CPT, CPT)])
        plsc.subcore_barrier()

        base = w * EPT

        @pl.loop(0, NBLK)
        def _(i):
            off = base + i * KB
            pltpu.sync_copy(dst_hbm.at[pl.ds(off, KB)], dst_v)
            pltpu.sync_copy(typ_hbm.at[pl.ds(off, KB)], typ_v)
            for j in range(KB // 16):
                sl = pl.ds(j * 16, 16)
                key_v[sl] = dst_v[sl] * R + typ_v[sl]
            pltpu.sync_copy(ones_v, acc_sh.at[key_v], add=True)

        plsc.subcore_barrier()
        pltpu.sync_copy(acc_sh.at[pl.ds(s * CPT * 2, CPT * 2)],
                        out_hbm.at[c, pl.ds(s * CPT * 2, CPT * 2)])

    return k(dst, typ)


def _inv_counts(cnt):
    @pl.kernel(
        out_type=jax.ShapeDtypeStruct((CPP,), jnp.float32),
        mesh=_mesh,
        scratch_types=[
            pltpu.VMEM((CPT,), jnp.float32),
            pltpu.VMEM((CPT,), jnp.float32),
        ],
    )
    def k(cnt_hbm, out_hbm, a_v, b_v):
        w = lax.axis_index("c") * NSUB + lax.axis_index("s")
        base = w * CPT
        pltpu.sync_copy(cnt_hbm.at[0, pl.ds(base, CPT)], a_v)
        pltpu.sync_copy(cnt_hbm.at[1, pl.ds(base, CPT)], b_v)

        @pl.loop(0, CPT, step=16)
        def _(i):
            sl = pl.ds(i, 16)
            a_v[sl] = 1.0 / jnp.maximum(a_v[sl] + b_v[sl], 1.0)

        pltpu.sync_copy(a_v, out_hbm.at[pl.ds(base, CPT)])

    return k(cnt)


def _edge_aggregate(pk, inv, yflat):
    @pl.kernel(
        out_type=jax.ShapeDtypeStruct((NSC, N, D), jnp.float32),
        mesh=_mesh,
        compiler_params=_sc_params,
        scratch_types=(
            [pltpu.VMEM((3, KB), jnp.int32)] * 3
            + [pltpu.VMEM((KB,), jnp.int32)] * 3
            + [pltpu.VMEM((KB,), jnp.int32)] * 3
            + [pltpu.VMEM((KB,), jnp.int32)] * 3
            + [pltpu.VMEM((KB + 16,), jnp.float32)] * 3
            + [pltpu.VMEM((KB, D), jnp.float32)] * 3
            + [
                pltpu.VMEM((ZR, D), jnp.float32),
                pltpu.VMEM_SHARED((N, D), jnp.float32),
            ]
            + [pltpu.SemaphoreType.DMA] * 12
        ),
    )
    def k(pk_hbm, inv_hbm, y_hbm, out_hbm,
          pk0, pk1, pk2, k1a, k1b, k1c, k2a, k2b, k2c, da, db, dc,
          wa, wb, wc, ra, rb, rc, z_v, acc_sh,
          sp0, sp1, sp2, sr0, sr1, sr2, si0, si1, si2, ss0, ss1, ss2):
        c = lax.axis_index("c")
        s = lax.axis_index("s")
        w = c * NSUB + s

        _zero_buf_rows(z_v, ZR)

        @pl.loop(s, N // ZR, step=NSUB)
        def _(g):
            pltpu.sync_copy(z_v, acc_sh.at[pl.ds(g * ZR, ZR)])

        plsc.subcore_barrier()

        bufs = ((pk0, k1a, k2a, da, wa, ra, sp0, sr0, si0, ss0),
                (pk1, k1b, k2b, db, wb, rb, sp1, sr1, si1, ss1),
                (pk2, k1c, k2c, dc, wc, rc, sp2, sr2, si2, ss2))
        base = w * NBLK

        def stage_a(i, t):
            pk_v, _, _, _, _, _, semp, _, _, _ = bufs[t]
            pltpu.async_copy(pk_hbm.at[base + i], pk_v, semp)

        def stage_b(i, t):
            pk_v, k1_v, k2_v, d_v, w_v, rows_v, semp, semr, semi, sems = \
                bufs[t]
            pltpu.make_async_copy(pk_hbm.at[base + i], pk_v, semp).wait()
            if isinstance(i, int):
                if i >= 3:
                    pltpu.make_async_copy(rows_v, acc_sh.at[d_v], sems).wait()
            else:
                @pl.when(i >= 3)
                def _():
                    pltpu.make_async_copy(rows_v, acc_sh.at[d_v], sems).wait()

            for j in range(KB // 16):
                sl = pl.ds(j * 16, 16)
                t16 = pk_v[2, sl]
                d16 = pk_v[1, sl]
                k1_v[sl] = t16 * N + pk_v[0, sl]
                k2_v[sl] = d16 * R + t16
                d_v[sl] = d16

            pltpu.async_copy(y_hbm.at[k1_v], rows_v, semr)
            pltpu.async_copy(inv_hbm.at[k2_v], w_v.at[pl.ds(16, KB)], semi)

        def stage_c(t):
            pk_v, k1_v, k2_v, d_v, w_v, rows_v, semp, semr, semi, sems = \
                bufs[t]
            pltpu.make_async_copy(y_hbm.at[k1_v], rows_v, semr).wait()
            pltpu.make_async_copy(inv_hbm.at[k2_v], w_v.at[pl.ds(16, KB)],
                                  semi).wait()
            @plsc.parallel_loop(0, KB, unroll=8)
            def _(e):
                wv = plsc.load_gather(
                    w_v, [jnp.full((16,), e + 16, jnp.int32)])
                for j in range(D // 16):
                    sl = pl.ds(j * 16, 16)
                    rows_v[e, sl] = rows_v[e, sl] * wv

            pltpu.async_copy(rows_v, acc_sh.at[d_v], sems, add=True)

        stage_a(0, 0)
        stage_a(1, 1)
        stage_b(0, 0)

        @pl.loop(0, (NBLK - 2) // 3)
        def _(p):
            i = 3 * p
            for q in range(3):
                stage_b(i + q + 1, (q + 1) % 3)
                stage_c(q)
                stage_a(i + q + 2, (q + 2) % 3)

        stage_b(NBLK - 1, (NBLK - 1) % 3)
        stage_c((NBLK - 2) % 3)
        stage_c((NBLK - 1) % 3)
        for t in range(3):
            pk_v, _, _, d_v, _, rows_v, _, _, _, sems = bufs[t]
            pltpu.make_async_copy(rows_v, acc_sh.at[d_v], sems).wait()

        plsc.subcore_barrier()

        @pl.loop(s, N // ZR, step=NSUB)
        def _(g):
            pltpu.sync_copy(acc_sh.at[pl.ds(g * ZR, ZR)],
                            out_hbm.at[c, pl.ds(g * ZR, ZR)])

    return k(pk, inv, yflat)


BN = 400


def _y_from_x(x, wall):
    def body(x_ref, w_ref, o_ref):
        res = jnp.dot(x_ref[...].astype(jnp.bfloat16), w_ref[...],
                      preferred_element_type=jnp.float32)
        for r in range(RP):
            o_ref[r] = res[:, r * D:(r + 1) * D]

    return pl.pallas_call(
        body,
        grid=(N // BN,),
        in_specs=[
            pl.BlockSpec((BN, D), lambda i: (i, 0)),
            pl.BlockSpec((D, RP * D), lambda i: (0, 0)),
        ],
        out_specs=pl.BlockSpec((RP, BN, D), lambda i: (0, i, 0)),
        out_shape=jax.ShapeDtypeStruct((RP, N, D), jnp.float32),
    )(x, wall)


def _y_from_agg(agg0, agg1, yprev, bias, wall):
    def body(a0_ref, a1_ref, yr_ref, b_ref, w_ref, o_ref):
        h = a0_ref[...] + a1_ref[...] + yr_ref[...] + b_ref[0][None, :]
        res = jnp.dot(h.astype(jnp.bfloat16), w_ref[...],
                      preferred_element_type=jnp.float32)
        for r in range(RP):
            o_ref[r] = res[:, r * D:(r + 1) * D]

    return pl.pallas_call(
        body,
        grid=(N // BN,),
        in_specs=[
            pl.BlockSpec((BN, D), lambda i: (i, 0)),
            pl.BlockSpec((BN, D), lambda i: (i, 0)),
            pl.BlockSpec((BN, D), lambda i: (i, 0)),
            pl.BlockSpec((1, D), lambda i: (0, 0)),
            pl.BlockSpec((D, RP * D), lambda i: (0, 0)),
        ],
        out_specs=pl.BlockSpec((RP, BN, D), lambda i: (0, i, 0)),
        out_shape=jax.ShapeDtypeStruct((RP, N, D), jnp.float32),
    )(agg0, agg1, yprev, bias, wall)


def _batch_pool(agg0, agg1, yprev, bias, bid3):
    def body(a0_ref, a1_ref, yr_ref, b_ref, bid_ref, o_ref):
        h = a0_ref[...] + a1_ref[...] + yr_ref[...] + b_ref[0][None, :]
        bb = bid_ref[0, 0, :]
        oh = (lax.broadcasted_iota(jnp.int32, (B, BN), 0)
              == bb[None, :]).astype(jnp.float32)

        @pl.when(pl.program_id(0) == 0)
        def _():
            o_ref[...] = jnp.zeros_like(o_ref)

        o_ref[...] += jnp.dot(oh, h, preferred_element_type=jnp.float32)

    return pl.pallas_call(
        body,
        grid=(N // BN,),
        in_specs=[
            pl.BlockSpec((BN, D), lambda i: (i, 0)),
            pl.BlockSpec((BN, D), lambda i: (i, 0)),
            pl.BlockSpec((BN, D), lambda i: (i, 0)),
            pl.BlockSpec((1, D), lambda i: (0, 0)),
            pl.BlockSpec((1, 1, BN), lambda i: (i, 0, 0)),
        ],
        out_specs=pl.BlockSpec((B, D), lambda i: (0, 0)),
        out_shape=jax.ShapeDtypeStruct((B, D), jnp.float32),
    )(agg0, agg1, yprev, bias, bid3)


def kernel(edge_index, edge_type, entity_ids, batch_ids, node_embeddings,
           W1, root1, b1, W2, root2, b2):
    src = edge_index[0]
    dst = edge_index[1]
    typ = edge_type.astype(jnp.int32)
    pk = jnp.stack([src, dst, typ]).reshape(3, E // KB, KB).transpose(1, 0, 2)

    x = _gather_x(node_embeddings, entity_ids.astype(jnp.int32))
    cnt = _edge_counts(dst, typ)
    inv = _inv_counts(cnt)

    w1a = jnp.concatenate([W1, root1[None]], axis=0)
    w2a = jnp.concatenate([W2, root2[None]], axis=0)
    w1f = w1a.transpose(1, 0, 2).reshape(D, RP * D).astype(jnp.bfloat16)
    w2f = w2a.transpose(1, 0, 2).reshape(D, RP * D).astype(jnp.bfloat16)

    y1 = _y_from_x(x, w1f)
    agg1 = _edge_aggregate(pk, inv, y1.reshape(RP * N, D))
    y2 = _y_from_agg(agg1[0], agg1[1], y1[R], b1.reshape(1, D), w2f)
    agg2 = _edge_aggregate(pk, inv, y2.reshape(RP * N, D))
    out = _batch_pool(agg2[0], agg2[1], y2[R], b2.reshape(1, D),
                      batch_ids.astype(jnp.int32).reshape(N // BN, 1, BN))
    return out


# speedup vs baseline: 39.6845x; 1.1992x over previous
import dataclasses

import jax
import jax.numpy as jnp
from jax import lax
from jax.experimental import pallas as pl
from jax.experimental.pallas import tpu as pltpu
from jax.experimental.pallas import tpu_sc as plsc

N = 10000
E = 320000
D = 128
R = 18
RP = R + 1
B = 128

NSC = 2
NSUB = 16
NW = NSC * NSUB
EPT = E // NW
KB = 80
NBLK = EPT // KB
CPP = 180224
CPT = CPP // NW
ZR = 80

_mesh = plsc.VectorSubcoreMesh(core_axis_name="c", subcore_axis_name="s")

_sc_params = pltpu.CompilerParams()
if "needs_layout_passes" in pltpu.CompilerParams.__dataclass_fields__:
    _sc_params = dataclasses.replace(_sc_params, needs_layout_passes=False)


def _zero_buf_rows(buf, rows):
    @pl.loop(0, rows)
    def _(i):
        for j in range(D // 16):
            buf[i, pl.ds(j * 16, 16)] = jnp.zeros((16,), jnp.float32)


def _gather_x(node_embeddings, entity_ids):
    @pl.kernel(
        out_type=jax.ShapeDtypeStruct((N, D), jnp.float32),
        mesh=_mesh,
        scratch_types=[
            pltpu.VMEM((KB,), jnp.int32),
            pltpu.VMEM((KB, D), jnp.float32),
            pltpu.SemaphoreType.DMA,
        ],
    )
    def k(tab_hbm, eid_hbm, x_hbm, idx_v, rows_v, sem):
        w = lax.axis_index("c") * NSUB + lax.axis_index("s")

        @pl.loop(w, N // KB, step=NW)
        def _(b):
            off = b * KB
            pltpu.sync_copy(eid_hbm.at[pl.ds(off, KB)], idx_v)
            pltpu.async_copy(tab_hbm.at[idx_v], rows_v, sem).wait()
            pltpu.sync_copy(rows_v, x_hbm.at[pl.ds(off, KB)])

    return k(node_embeddings, entity_ids)


def _edge_counts(pk, x):
    @pl.kernel(
        out_type=jax.ShapeDtypeStruct((NSC, CPP), jnp.float32),
        mesh=_mesh,
        compiler_params=_sc_params,
        scratch_types=[
            pltpu.VMEM((3, KB), jnp.int32),
            pltpu.VMEM((3, KB), jnp.int32),
            pltpu.VMEM((KB,), jnp.int32),
            pltpu.VMEM((KB,), jnp.int32),
            pltpu.VMEM((KB,), jnp.float32),
            pltpu.VMEM((CPT,), jnp.float32),
            pltpu.VMEM_SHARED((CPP,), jnp.float32),
            pltpu.SemaphoreType.DMA,
            pltpu.SemaphoreType.DMA,
            pltpu.SemaphoreType.DMA,
            pltpu.SemaphoreType.DMA,
        ],
    )
    def k(pk_hbm, x_hbm, out_hbm, pk0, pk1, key0, key1, ones_v, z_v, acc_sh,
          sp0, sp1, ss0, ss1):
        del x_hbm
        c = lax.axis_index("c")
        s = lax.axis_index("s")
        w = c * NSUB + s

        for j in range(KB // 16):
            ones_v[pl.ds(j * 16, 16)] = jnp.ones((16,), jnp.float32)

        @pl.loop(0, CPT, step=16)
        def _(i):
            z_v[pl.ds(i, 16)] = jnp.zeros((16,), jnp.float32)

        pltpu.sync_copy(z_v, acc_sh.at[pl.ds(s * CPT * 2, CPT)])
        pltpu.sync_copy(z_v, acc_sh.at[pl.ds(s * CPT * 2 + CPT, CPT)])
        plsc.subcore_barrier()

        bufs = ((pk0, key0, sp0, ss0), (pk1, key1, sp1, ss1))
        base = w * NBLK

        def fire_pk(i, t):
            pk_v, _, semp, _ = bufs[t]
            pltpu.async_copy(pk_hbm.at[base + i], pk_v, semp)

        def count_block(i, t, first):
            pk_v, key_v, semp, sems = bufs[t]
            pltpu.make_async_copy(pk_hbm.at[base + i], pk_v, semp).wait()
            if not first:
                pltpu.make_async_copy(ones_v, acc_sh.at[key_v], sems).wait()
            for j in range(KB // 16):
                sl = pl.ds(j * 16, 16)
                key_v[sl] = pk_v[1, sl] * R + pk_v[2, sl]
            pltpu.async_copy(ones_v, acc_sh.at[key_v], sems, add=True)

        fire_pk(0, 0)
        fire_pk(1, 1)
        count_block(0, 0, True)
        fire_pk(2, 0)
        count_block(1, 1, True)
        fire_pk(3, 1)

        @pl.loop(0, (NBLK - 5) // 2)
        def _(p):
            i = 2 * p
            count_block(i + 2, 0, False)
            fire_pk(i + 4, 0)
            count_block(i + 3, 1, False)
            fire_pk(i + 5, 1)

        count_block(NBLK - 3, 0, False)
        fire_pk(NBLK - 1, 0)
        count_block(NBLK - 2, 1, False)
        count_block(NBLK - 1, 0, False)
        for t in range(2):
            _, key_v, _, sems = bufs[t]
            pltpu.make_async_copy(ones_v, acc_sh.at[key_v], sems).wait()

        plsc.subcore_barrier()
        pltpu.sync_copy(acc_sh.at[pl.ds(s * CPT * 2, CPT * 2)],
                        out_hbm.at[c, pl.ds(s * CPT * 2, CPT * 2)])

    return k(pk, x)


def _inv_counts(cnt):
    @pl.kernel(
        out_type=jax.ShapeDtypeStruct((CPP,), jnp.float32),
        mesh=_mesh,
        scratch_types=[
            pltpu.VMEM((CPT,), jnp.float32),
            pltpu.VMEM((CPT,), jnp.float32),
        ],
    )
    def k(cnt_hbm, out_hbm, a_v, b_v):
        w = lax.axis_index("c") * NSUB + lax.axis_index("s")
        base = w * CPT
        pltpu.sync_copy(cnt_hbm.at[0, pl.ds(base, CPT)], a_v)
        pltpu.sync_copy(cnt_hbm.at[1, pl.ds(base, CPT)], b_v)

        @pl.loop(0, CPT, step=16)
        def _(i):
            sl = pl.ds(i, 16)
            a_v[sl] = 1.0 / jnp.maximum(a_v[sl] + b_v[sl], 1.0)

        pltpu.sync_copy(a_v, out_hbm.at[pl.ds(base, CPT)])

    return k(cnt)


def _edge_aggregate(pk, inv, yflat):
    @pl.kernel(
        out_type=jax.ShapeDtypeStruct((NSC, N, D), jnp.float32),
        mesh=_mesh,
        compiler_params=_sc_params,
        scratch_types=(
            [pltpu.VMEM((3, KB), jnp.int32)] * 3
            + [pltpu.VMEM((KB,), jnp.int32)] * 3
            + [pltpu.VMEM((KB,), jnp.int32)] * 3
            + [pltpu.VMEM((KB,), jnp.int32)] * 3
            + [pltpu.VMEM((KB + 16,), jnp.float32)] * 3
            + [pltpu.VMEM((KB, D), jnp.float32)] * 3
            + [
                pltpu.VMEM((ZR, D), jnp.float32),
                pltpu.VMEM_SHARED((N, D), jnp.float32),
            ]
            + [pltpu.SemaphoreType.DMA] * 12
        ),
    )
    def k(pk_hbm, inv_hbm, y_hbm, out_hbm,
          pk0, pk1, pk2, k1a, k1b, k1c, k2a, k2b, k2c, da, db, dc,
          wa, wb, wc, ra, rb, rc, z_v, acc_sh,
          sp0, sp1, sp2, sr0, sr1, sr2, si0, si1, si2, ss0, ss1, ss2):
        c = lax.axis_index("c")
        s = lax.axis_index("s")
        w = c * NSUB + s

        _zero_buf_rows(z_v, ZR)

        @pl.loop(s, N // ZR, step=NSUB)
        def _(g):
            pltpu.sync_copy(z_v, acc_sh.at[pl.ds(g * ZR, ZR)])

        plsc.subcore_barrier()

        bufs = ((pk0, k1a, k2a, da, wa, ra, sp0, sr0, si0, ss0),
                (pk1, k1b, k2b, db, wb, rb, sp1, sr1, si1, ss1),
                (pk2, k1c, k2c, dc, wc, rc, sp2, sr2, si2, ss2))
        base = w * NBLK

        def stage_a(i, t):
            pk_v, _, _, _, _, _, semp, _, _, _ = bufs[t]
            pltpu.async_copy(pk_hbm.at[base + i], pk_v, semp)

        def stage_b(i, t):
            pk_v, k1_v, k2_v, d_v, w_v, rows_v, semp, semr, semi, sems = \
                bufs[t]
            pltpu.make_async_copy(pk_hbm.at[base + i], pk_v, semp).wait()
            if isinstance(i, int):
                if i >= 3:
                    pltpu.make_async_copy(rows_v, acc_sh.at[d_v], sems).wait()
            else:
                @pl.when(i >= 3)
                def _():
                    pltpu.make_async_copy(rows_v, acc_sh.at[d_v], sems).wait()

            for j in range(KB // 16):
                sl = pl.ds(j * 16, 16)
                t16 = pk_v[2, sl]
                d16 = pk_v[1, sl]
                k1_v[sl] = t16 * N + pk_v[0, sl]
                k2_v[sl] = d16 * R + t16
                d_v[sl] = d16

            pltpu.async_copy(y_hbm.at[k1_v], rows_v, semr)
            pltpu.async_copy(inv_hbm.at[k2_v], w_v.at[pl.ds(16, KB)], semi)

        def stage_c(t):
            pk_v, k1_v, k2_v, d_v, w_v, rows_v, semp, semr, semi, sems = \
                bufs[t]
            pltpu.make_async_copy(y_hbm.at[k1_v], rows_v, semr).wait()
            pltpu.make_async_copy(inv_hbm.at[k2_v], w_v.at[pl.ds(16, KB)],
                                  semi).wait()
            @plsc.parallel_loop(0, KB, unroll=8)
            def _(e):
                wv = plsc.load_gather(
                    w_v, [jnp.full((16,), e + 16, jnp.int32)])
                for j in range(D // 16):
                    sl = pl.ds(j * 16, 16)
                    rows_v[e, sl] = rows_v[e, sl] * wv

            pltpu.async_copy(rows_v, acc_sh.at[d_v], sems, add=True)

        stage_a(0, 0)
        stage_a(1, 1)
        stage_b(0, 0)

        @pl.loop(0, (NBLK - 2) // 3)
        def _(p):
            i = 3 * p
            for q in range(3):
                stage_b(i + q + 1, (q + 1) % 3)
                stage_c(q)
                stage_a(i + q + 2, (q + 2) % 3)

        stage_b(NBLK - 1, (NBLK - 1) % 3)
        stage_c((NBLK - 2) % 3)
        stage_c((NBLK - 1) % 3)
        for t in range(3):
            pk_v, _, _, d_v, _, rows_v, _, _, _, sems = bufs[t]
            pltpu.make_async_copy(rows_v, acc_sh.at[d_v], sems).wait()

        plsc.subcore_barrier()

        @pl.loop(s, N // ZR, step=NSUB)
        def _(g):
            pltpu.sync_copy(acc_sh.at[pl.ds(g * ZR, ZR)],
                            out_hbm.at[c, pl.ds(g * ZR, ZR)])

    return k(pk, inv, yflat)


BN = 400


def _y_from_x(x, wall):
    def body(x_ref, w_ref, o_ref):
        res = jnp.dot(x_ref[...].astype(jnp.bfloat16), w_ref[...],
                      preferred_element_type=jnp.float32)
        for r in range(RP):
            o_ref[r] = res[:, r * D:(r + 1) * D]

    return pl.pallas_call(
        body,
        grid=(N // BN,),
        in_specs=[
            pl.BlockSpec((BN, D), lambda i: (i, 0)),
            pl.BlockSpec((D, RP * D), lambda i: (0, 0)),
        ],
        out_specs=pl.BlockSpec((RP, BN, D), lambda i: (0, i, 0)),
        out_shape=jax.ShapeDtypeStruct((RP, N, D), jnp.float32),
    )(x, wall)


def _y_from_agg(agg0, agg1, yprev, bias, wall):
    def body(a0_ref, a1_ref, yr_ref, b_ref, w_ref, o_ref):
        h = a0_ref[...] + a1_ref[...] + yr_ref[...] + b_ref[0][None, :]
        res = jnp.dot(h.astype(jnp.bfloat16), w_ref[...],
                      preferred_element_type=jnp.float32)
        for r in range(RP):
            o_ref[r] = res[:, r * D:(r + 1) * D]

    return pl.pallas_call(
        body,
        grid=(N // BN,),
        in_specs=[
            pl.BlockSpec((BN, D), lambda i: (i, 0)),
            pl.BlockSpec((BN, D), lambda i: (i, 0)),
            pl.BlockSpec((BN, D), lambda i: (i, 0)),
            pl.BlockSpec((1, D), lambda i: (0, 0)),
            pl.BlockSpec((D, RP * D), lambda i: (0, 0)),
        ],
        out_specs=pl.BlockSpec((RP, BN, D), lambda i: (0, i, 0)),
        out_shape=jax.ShapeDtypeStruct((RP, N, D), jnp.float32),
    )(agg0, agg1, yprev, bias, wall)


def _batch_pool(agg0, agg1, yprev, bias, bid3):
    def body(a0_ref, a1_ref, yr_ref, b_ref, bid_ref, o_ref):
        h = a0_ref[...] + a1_ref[...] + yr_ref[...] + b_ref[0][None, :]
        bb = bid_ref[0, 0, :]
        oh = (lax.broadcasted_iota(jnp.int32, (B, BN), 0)
              == bb[None, :]).astype(jnp.float32)

        @pl.when(pl.program_id(0) == 0)
        def _():
            o_ref[...] = jnp.zeros_like(o_ref)

        o_ref[...] += jnp.dot(oh, h, preferred_element_type=jnp.float32)

    return pl.pallas_call(
        body,
        grid=(N // BN,),
        in_specs=[
            pl.BlockSpec((BN, D), lambda i: (i, 0)),
            pl.BlockSpec((BN, D), lambda i: (i, 0)),
            pl.BlockSpec((BN, D), lambda i: (i, 0)),
            pl.BlockSpec((1, D), lambda i: (0, 0)),
            pl.BlockSpec((1, 1, BN), lambda i: (i, 0, 0)),
        ],
        out_specs=pl.BlockSpec((B, D), lambda i: (0, 0)),
        out_shape=jax.ShapeDtypeStruct((B, D), jnp.float32),
    )(agg0, agg1, yprev, bias, bid3)


def kernel(edge_index, edge_type, entity_ids, batch_ids, node_embeddings,
           W1, root1, b1, W2, root2, b2):
    src = edge_index[0]
    dst = edge_index[1]
    typ = edge_type.astype(jnp.int32)
    pk = jnp.stack([src, dst, typ]).reshape(3, E // KB, KB).transpose(1, 0, 2)

    x = _gather_x(node_embeddings, entity_ids.astype(jnp.int32))
    cnt = _edge_counts(pk, x)
    inv = _inv_counts(cnt)

    w1a = jnp.concatenate([W1, root1[None]], axis=0)
    w2a = jnp.concatenate([W2, root2[None]], axis=0)
    w1f = w1a.transpose(1, 0, 2).reshape(D, RP * D).astype(jnp.bfloat16)
    w2f = w2a.transpose(1, 0, 2).reshape(D, RP * D).astype(jnp.bfloat16)

    y1 = _y_from_x(x, w1f)
    agg1 = _edge_aggregate(pk, inv, y1.reshape(RP * N, D))
    y2 = _y_from_agg(agg1[0], agg1[1], y1[R], b1.reshape(1, D), w2f)
    agg2 = _edge_aggregate(pk, inv, y2.reshape(RP * N, D))
    out = _batch_pool(agg2[0], agg2[1], y2[R], b2.reshape(1, D),
                      batch_ids.astype(jnp.int32).reshape(N // BN, 1, BN))
    return out
